# Initial kernel scaffold; baseline (speedup 1.0000x reference)
import jax
import jax.numpy as jnp
from jax.experimental import pallas as pl


def kernel(op_feats, nconfig_feats, emb, pre_W1, pre_b1, pre_W2, pre_b2, gc1_W1, gc1_b1, gc1_W2, gc1_b2, gc2_W1, gc2_b1, gc2_W2, gc2_b2, post_W1, post_W2, op_ids, selected, feed_edges, sampled_feed_edges, config_dst, sampled_config_dst, graph_id_op, graph_id_config):
    raise NotImplementedError("write your pallas kernel here")


# speedup vs baseline: 79.1967x; 79.1967x over previous
import jax
import jax.numpy as jnp
from jax import lax
from jax.experimental import pallas as pl
from jax.experimental.pallas import tpu as pltpu
from jax.experimental.pallas import tpu_sc as plsc

N = 10000
C = 5
HID = 32
RW = C * HID
CFD = 18
CFW = 96
G = 8
VOCAB = 120
OPF = 140
EF = 320000
ES = 160000
NCF = 1000
CH = 128
NW = 32
EFP = 327680
ESP = 163840
DFP = 655360
DSP = 327680
NP = 10112
NTILES = 16
STRIPE = NP // NTILES
ALPHA = 0.2
BLK = 400
NBLK = N // BLK

_HI = lax.Precision.HIGHEST
_f32 = jnp.float32


def _leaky(x):
    return jnp.where(x > 0, x, ALPHA * x)


def _dot(a, b):
    return jnp.dot(a, b, precision=_HI, preferred_element_type=_f32)


def _dot_t(a, b):
    return lax.dot_general(a, b, (((0,), (0,)), ((), ())),
                           precision=_HI, preferred_element_type=_f32)


def _l2n(y):
    return y * lax.rsqrt(jnp.maximum(jnp.sum(y * y, axis=-1, keepdims=True),
                                     1e-12))


def _sc_pro_body(idxF, idxS, cdF, cdS, P, z96, z16,
                 cfF, cfS, degF, degS,
                 acc_cf, acc_deg, pbuf, ones, idxb, cdv):
    c = lax.axis_index("c")
    s = lax.axis_index("s")
    wid = c * NTILES + s
    row0 = s * STRIPE

    def fill(j, carry):
        ones[j] = jnp.ones((16,), _f32)
        return carry
    lax.fori_loop(0, CH, fill, 0)

    pltpu.sync_copy(z96.at[pl.ds(row0, STRIPE)], acc_cf.at[pl.ds(row0, STRIPE)])
    plsc.subcore_barrier()

    @pl.when((c == 0) & (s == 0))
    def _():
        pltpu.sync_copy(cdF, cdv)

    @pl.when((c == 1) & (s == 0))
    def _():
        pltpu.sync_copy(cdS, cdv)

    @pl.when(s == 0)
    def _():
        def cbody(j, carry):
            pltpu.sync_copy(P.at[pl.ds(j * CH, CH)], pbuf)
            pltpu.sync_copy(pbuf, acc_cf.at[cdv.at[j]], add=True)
            return carry
        lax.fori_loop(0, NCF // CH + 1, cbody, 0)

    plsc.subcore_barrier()

    @pl.when(c == 0)
    def _():
        pltpu.sync_copy(acc_cf.at[pl.ds(row0, STRIPE)],
                        cfF.at[pl.ds(row0, STRIPE)])

    @pl.when(c == 1)
    def _():
        pltpu.sync_copy(acc_cf.at[pl.ds(row0, STRIPE)],
                        cfS.at[pl.ds(row0, STRIPE)])

    for idx_hbm, nch, out in ((idxF, DFP // CH // NW, degF),
                              (idxS, DSP // CH // NW, degS)):
        pltpu.sync_copy(z16.at[pl.ds(row0, STRIPE)],
                        acc_deg.at[pl.ds(row0, STRIPE)])
        plsc.subcore_barrier()
        pltpu.sync_copy(idx_hbm.at[pl.ds(wid * nch, nch)],
                        idxb.at[pl.ds(0, nch)])

        def body(j, carry):
            pltpu.sync_copy(ones, acc_deg.at[idxb.at[j]], add=True)
            return carry
        lax.fori_loop(0, nch, body, 0)
        plsc.subcore_barrier()
        pltpu.sync_copy(acc_deg.at[pl.ds(row0, STRIPE)],
                        out.at[c].at[pl.ds(row0, STRIPE)])
        plsc.subcore_barrier()


GCH = 32


def _sc_adj_body(zsF, zsS, sF, dF, sS, dS, z160,
                 pF, pS,
                 acc, sidx, didx, rows, rows2, sem):
    c = lax.axis_index("c")
    s = lax.axis_index("s")
    wid = c * NTILES + s
    row0 = s * STRIPE
    for zs, si, di, epw, out in ((zsF, sF, dF, EFP // NW, pF),
                                 (zsS, sS, dS, ESP // NW, pS)):
        pltpu.sync_copy(z160.at[pl.ds(row0, STRIPE)],
                        acc.at[pl.ds(row0, STRIPE)])
        plsc.subcore_barrier()
        base_row = wid * (epw // GCH)

        def outer(g, carry):
            pltpu.sync_copy(si.at[pl.ds(base_row + g * GCH, GCH)], sidx)
            pltpu.sync_copy(di.at[pl.ds(base_row + g * GCH, GCH)], didx)

            def body(k, carry2):
                pltpu.async_copy(zs.at[sidx.at[k]], rows, sem).wait()
                pltpu.sync_copy(rows, acc.at[didx.at[k]], add=True)
                pltpu.async_copy(zs.at[didx.at[k]], rows2, sem).wait()
                pltpu.sync_copy(rows2, acc.at[sidx.at[k]], add=True)
                return carry2
            lax.fori_loop(0, GCH, body, 0)
            return carry
        lax.fori_loop(0, epw // GCH // GCH, outer, 0)
        plsc.subcore_barrier()
        pltpu.sync_copy(acc.at[pl.ds(row0, STRIPE)],
                        out.at[c].at[pl.ds(row0, STRIPE)])
        plsc.subcore_barrier()


def _sc_epi_body(x, cd, out, idxv, buf, sem):
    c = lax.axis_index("c")
    s = lax.axis_index("s")
    wid = c * NTILES + s
    pltpu.sync_copy(cd, idxv)
    pltpu.async_copy(x.at[idxv.at[wid]], buf, sem).wait()
    pltpu.sync_copy(buf, out.at[pl.ds(wid * 32, 32)])


def _tc_pre_body(opf, idsf, cfF, cfS, dgF, dgS,
                 emb, Wf, We, Wcp, b1, W2, b2, Wc1, Wx1,
                 x0F, x0S, zs1F, zs1S, disF, disS):
    oh = (idsf[...] == lax.broadcasted_iota(jnp.int32, (BLK, VOCAB), 1))
    e = _dot(oh.astype(_f32), emb[...])
    nfb = _dot(opf[...], Wf[...]) + _dot(e, We[...]) + b1[...]
    for cf, dg, x0o, zso, diso in ((cfF, dgF, x0F, zs1F, disF),
                                   (cfS, dgS, x0S, zs1S, disS)):
        dgv = dg[...]
        dis = lax.rsqrt(1.0 + dgv[0, :, 0:1] + dgv[1, :, 0:1])
        diso[...] = dis
        cfb = cf[...] * 100.0
        for k in range(C):
            cfc = cfb[:, k * CFD:(k + 1) * CFD]
            t = _leaky(_dot(cfc, Wcp[...]) + nfb)
            x0c = _leaky(_dot(t, W2[...]) + b2[...])
            z1 = _dot(cfc, Wc1[...]) + _dot(x0c, Wx1[...])
            x0o[:, k * HID:(k + 1) * HID] = x0c
            zso[:, k * HID:(k + 1) * HID] = dis * z1


def _tc_mid1_body(x0F, x0S, zsF, zsS, pF, pS, cfF, cfS, disF, disS,
                  b1, W2, b2, Wc2, Wx2,
                  x1F, x1S, zs2F, zs2S):
    for x0, zs, p, cf, dis, x1o, zs2o in (
            (x0F, zsF, pF, cfF, disF, x1F, zs2F),
            (x0S, zsS, pS, cfS, disS, x1S, zs2S)):
        d = dis[...]
        pv = p[...]
        a = d * (zs[...] + pv[0] + pv[1])
        cfb = cf[...] * 100.0
        x0v = x0[...]
        for k in range(C):
            sl = slice(k * HID, (k + 1) * HID)
            h = _dot(_leaky(a[:, sl] + b1[...]), W2[...]) + b2[...]
            x1c = x0v[:, sl] + _leaky(h)
            z2 = _dot(cfb[:, k * CFD:(k + 1) * CFD], Wc2[...]) + _dot(x1c, Wx2[...])
            x1o[:, sl] = x1c
            zs2o[:, sl] = d * z2


def _tc_mid2_body(x1F, x1S, zsF, zsS, pF, pS, disF, disS, self_f, gof,
                  b1, W2, b2,
                  x, pooled, counts):
    xs = []
    for x1, zs, p, dis in ((x1F, zsF, pF, disF), (x1S, zsS, pS, disS)):
        d = dis[...]
        pv = p[...]
        a = d * (zs[...] + pv[0] + pv[1])
        x1v = x1[...]
        cols = []
        for k in range(C):
            sl = slice(k * HID, (k + 1) * HID)
            h = _dot(_leaky(a[:, sl] + b1[...]), W2[...]) + b2[...]
            cols.append(x1v[:, sl] + _leaky(h))
        xs.append(jnp.concatenate(cols, axis=1))
    sel = self_f[...]
    xb = sel * xs[1] + (1.0 - sel) * xs[0]
    x[...] = xb
    oh = (gof[...] == lax.broadcasted_iota(jnp.int32, (BLK, G), 1)).astype(_f32)
    ps = _dot_t(oh, xb)
    cnt = _dot_t(oh, jnp.ones((BLK, 1), _f32))

    @pl.when(pl.program_id(0) == 0)
    def _():
        pooled[...] = ps
        counts[...] = cnt

    @pl.when(pl.program_id(0) != 0)
    def _():
        pooled[...] += ps
        counts[...] += cnt


def _tc_final_body(pooled, counts, cfgx, gidc, pW1, pW2, out):
    oh = (gidc[...] == lax.broadcasted_iota(jnp.int32, (1024, G), 1)).astype(_f32)
    pc = _dot_t(oh, cfgx[...])
    ps = pooled[...]
    pm = ps / jnp.maximum(counts[...], 1.0)
    cols = []
    for k in range(C):
        sl = slice(k * HID, (k + 1) * HID)
        h = jnp.concatenate([pm[:, sl], _l2n(ps[:, sl]), _l2n(pc[:, sl])],
                            axis=1)
        cols.append(_dot(_leaky(_dot(h, pW1[...])), pW2[...]))
    cols.append(jnp.zeros((G, 128 - C), _f32))
    out[...] = jnp.concatenate(cols, axis=1)


def kernel(op_feats, nconfig_feats, emb, pre_W1, pre_b1, pre_W2, pre_b2,
           gc1_W1, gc1_b1, gc1_W2, gc1_b2, gc2_W1, gc2_b1, gc2_W2, gc2_b2,
           post_W1, post_W2, op_ids, selected, feed_edges, sampled_feed_edges,
           config_dst, sampled_config_dst, graph_id_op, graph_id_config):
    i32 = jnp.int32
    mesh = plsc.VectorSubcoreMesh(core_axis_name="c", subcore_axis_name="s")
    scparams = pltpu.CompilerParams(use_tc_tiling_on_sc=False)

    def padidx(n):
        return N + (jnp.arange(n, dtype=i32) % 16)

    fe = feed_edges.astype(i32)
    se = sampled_feed_edges.astype(i32)
    sF = jnp.concatenate([fe[0], padidx(EFP - EF)]).reshape(EFP // GCH, GCH)
    dF = jnp.concatenate([fe[1], padidx(EFP - EF)]).reshape(EFP // GCH, GCH)
    sS = jnp.concatenate([se[0], padidx(ESP - ES)]).reshape(ESP // GCH, GCH)
    dS = jnp.concatenate([se[1], padidx(ESP - ES)]).reshape(ESP // GCH, GCH)
    idxF = jnp.concatenate([fe[0], fe[1], padidx(DFP - 2 * EF)]
                           ).reshape(DFP // CH, CH)
    idxS = jnp.concatenate([se[0], se[1], padidx(DSP - 2 * ES)]
                           ).reshape(DSP // CH, CH)
    cdF = jnp.concatenate([config_dst.astype(i32), padidx(24)]).reshape(8, CH)
    cdS = jnp.concatenate([sampled_config_dst.astype(i32), padidx(24)]
                          ).reshape(8, CH)
    cdpad = jnp.pad(config_dst.astype(i32), (0, 24)).reshape(32, 32)
    P = jnp.pad(jnp.reshape(nconfig_feats, (NCF, C * CFD)),
                ((0, 24), (0, CFW - C * CFD)))
    z160 = jnp.zeros((NP, RW), _f32)
    z96 = jnp.zeros((NP, CFW), _f32)
    z16 = jnp.zeros((NP, 16), _f32)
    idsf = op_ids.astype(i32)[:, None]
    self_f = selected.astype(_f32)[:, None]
    gof = graph_id_op.astype(i32)[:, None]
    gic = jnp.pad(graph_id_config.astype(i32), (0, 24),
                  constant_values=G)[:, None]
    Wcp, Wf, We = pre_W1[:CFD], pre_W1[CFD:CFD + OPF], pre_W1[CFD + OPF:]
    Wc1, Wx1 = gc1_W1[:CFD], gc1_W1[CFD:]
    Wc2, Wx2 = gc2_W1[:CFD], gc2_W1[CFD:]
    b_pre1 = pre_b1[None, :]
    b_pre2 = pre_b2[None, :]
    b_g11 = gc1_b1[None, :]
    b_g12 = gc1_b2[None, :]
    b_g21 = gc2_b1[None, :]
    b_g22 = gc2_b2[None, :]

    sds = jax.ShapeDtypeStruct

    sc_pro = pl.kernel(
        _sc_pro_body,
        out_type=[sds((NP, CFW), _f32), sds((NP, CFW), _f32),
                  sds((2, NP, 16), _f32), sds((2, NP, 16), _f32)],
        mesh=mesh,
        scratch_types=[pltpu.VMEM_SHARED((NP, CFW), _f32),
                       pltpu.VMEM_SHARED((NP, 16), _f32),
                       pltpu.VMEM((CH, CFW), _f32),
                       pltpu.VMEM((CH, 16), _f32),
                       pltpu.VMEM((DFP // CH // NW, CH), i32),
                       pltpu.VMEM((8, CH), i32)],
        compiler_params=scparams)
    cfF, cfS, degF, degS = sc_pro(idxF, idxS, cdF, cdS, P, z96, z16)

    def fullspec(shp):
        return pl.BlockSpec(shp, lambda i: tuple(0 for _ in shp))

    def blkspec(r):
        return pl.BlockSpec((BLK, r), lambda i: (i, 0))

    degspec = pl.BlockSpec((2, BLK, 16), lambda i: (0, i, 0))

    x0F, x0S, zs1F, zs1S, disF, disS = pl.pallas_call(
        _tc_pre_body,
        grid=(NBLK,),
        in_specs=[blkspec(OPF), blkspec(1), blkspec(CFW), blkspec(CFW),
                  degspec, degspec,
                  fullspec((VOCAB, HID)), fullspec((OPF, HID)),
                  fullspec((HID, HID)), fullspec((CFD, HID)),
                  fullspec((1, HID)), fullspec((HID, HID)),
                  fullspec((1, HID)), fullspec((CFD, HID)),
                  fullspec((HID, HID))],
        out_specs=[blkspec(RW), blkspec(RW), blkspec(RW), blkspec(RW),
                   blkspec(1), blkspec(1)],
        out_shape=[sds((NP, RW), _f32), sds((NP, RW), _f32),
                   sds((NP, RW), _f32), sds((NP, RW), _f32),
                   sds((NP, 1), _f32), sds((NP, 1), _f32)],
    )(op_feats, idsf, cfF, cfS, degF, degS,
      emb, Wf, We, Wcp, b_pre1, pre_W2, b_pre2, Wc1, Wx1)

    adj = pl.kernel(
        _sc_adj_body,
        out_type=[sds((2, NP, RW), _f32), sds((2, NP, RW), _f32)],
        mesh=mesh,
        scratch_types=[pltpu.VMEM_SHARED((NP, RW), _f32),
                       pltpu.VMEM((GCH, GCH), i32),
                       pltpu.VMEM((GCH, GCH), i32),
                       pltpu.VMEM((GCH, RW), _f32),
                       pltpu.VMEM((GCH, RW), _f32),
                       pltpu.SemaphoreType.DMA],
        compiler_params=scparams)
    p1F, p1S = adj(zs1F, zs1S, sF, dF, sS, dS, z160)

    pspec = pl.BlockSpec((2, BLK, RW), lambda i: (0, i, 0))
    x1F, x1S, zs2F, zs2S = pl.pallas_call(
        _tc_mid1_body,
        grid=(NBLK,),
        in_specs=[blkspec(RW), blkspec(RW), blkspec(RW), blkspec(RW),
                  pspec, pspec, blkspec(CFW), blkspec(CFW),
                  blkspec(1), blkspec(1),
                  fullspec((1, HID)), fullspec((HID, HID)),
                  fullspec((1, HID)), fullspec((CFD, HID)),
                  fullspec((HID, HID))],
        out_specs=[blkspec(RW), blkspec(RW), blkspec(RW), blkspec(RW)],
        out_shape=[sds((NP, RW), _f32), sds((NP, RW), _f32),
                   sds((NP, RW), _f32), sds((NP, RW), _f32)],
    )(x0F, x0S, zs1F, zs1S, p1F, p1S, cfF, cfS, disF, disS,
      b_g11, gc1_W2, b_g12, Wc2, Wx2)

    p2F, p2S = adj(zs2F, zs2S, sF, dF, sS, dS, z160)

    x, pooled, counts = pl.pallas_call(
        _tc_mid2_body,
        grid=(NBLK,),
        in_specs=[blkspec(RW), blkspec(RW), blkspec(RW), blkspec(RW),
                  pspec, pspec, blkspec(1), blkspec(1), blkspec(1), blkspec(1),
                  fullspec((1, HID)), fullspec((HID, HID)),
                  fullspec((1, HID))],
        out_specs=[blkspec(RW),
                   pl.BlockSpec((G, RW), lambda i: (0, 0)),
                   pl.BlockSpec((G, 1), lambda i: (0, 0))],
        out_shape=[sds((NP, RW), _f32), sds((G, RW), _f32), sds((G, 1), _f32)],
    )(x1F, x1S, zs2F, zs2S, p2F, p2S, disF, disS, self_f, gof,
      b_g21, gc2_W2, b_g22)

    sc_epi = pl.kernel(
        _sc_epi_body,
        out_type=sds((1024, RW), _f32),
        mesh=mesh,
        scratch_types=[pltpu.VMEM((32, 32), i32),
                       pltpu.VMEM((32, RW), _f32),
                       pltpu.SemaphoreType.DMA],
        compiler_params=scparams)
    cfgx = sc_epi(x, cdpad)

    out8 = pl.pallas_call(
        _tc_final_body,
        grid=(1,),
        in_specs=[fullspec((G, RW)), fullspec((G, 1)),
                  fullspec((1024, RW)), fullspec((1024, 1)),
                  fullspec((3 * HID, HID)), fullspec((HID, 1))],
        out_specs=[fullspec((G, 128))],
        out_shape=[sds((G, 128), _f32)],
    )(pooled, counts, cfgx, gic, post_W1, post_W2)[0]

    return out8[:, :C]


# speedup vs baseline: 145.1400x; 1.8327x over previous
import jax
import jax.numpy as jnp
from jax import lax
from jax.experimental import pallas as pl
from jax.experimental.pallas import tpu as pltpu
from jax.experimental.pallas import tpu_sc as plsc

N = 10000
C = 5
HID = 32
RW = C * HID
YD = 50
YW = 256
HW = 128
CFD = 18
CFW = 96
G = 8
VOCAB = 120
OPF = 140
EF = 320000
ES = 160000
NCF = 1000
CH = 128
NW = 32
EFP = 327680
ESP = 163840
DFP = 655360
DSP = 327680
NP = 10112
NTILES = 16
STRIPE = NP // NTILES
ALPHA = 0.2
BLK = 400
NBLK = N // BLK
GCH = 128
IBLK = 16

_f32 = jnp.float32


def _leaky(x):
    return jnp.where(x > 0, x, ALPHA * x)


def _dot(a, b):
    return jnp.dot(a, b, precision=lax.Precision.DEFAULT,
                   preferred_element_type=_f32)


def _dotx(a, b):
    return jnp.dot(a, b, precision=lax.Precision.HIGHEST,
                   preferred_element_type=_f32)


def _dotx_t(a, b):
    return lax.dot_general(a, b, (((0,), (0,)), ((), ())),
                           precision=lax.Precision.HIGHEST,
                           preferred_element_type=_f32)


def _l2n(y):
    return y * lax.rsqrt(jnp.maximum(jnp.sum(y * y, axis=-1, keepdims=True),
                                     1e-12))


def _sc_pro_body(idxF, idxS, cdF, cdS, P, z96, z16,
                 cfF, cfS, degF, degS,
                 acc_cf, acc_deg, pbuf, ones, idxb, cdv):
    c = lax.axis_index("c")
    s = lax.axis_index("s")
    wid = c * NTILES + s
    row0 = s * STRIPE

    def fill(j, carry):
        ones[j] = jnp.ones((16,), _f32)
        return carry
    lax.fori_loop(0, CH, fill, 0)

    pltpu.sync_copy(z96.at[pl.ds(row0, STRIPE)], acc_cf.at[pl.ds(row0, STRIPE)])
    plsc.subcore_barrier()

    @pl.when((c == 0) & (s == 0))
    def _():
        pltpu.sync_copy(cdF, cdv)

    @pl.when((c == 1) & (s == 0))
    def _():
        pltpu.sync_copy(cdS, cdv)

    @pl.when(s == 0)
    def _():
        def cbody(j, carry):
            pltpu.sync_copy(P.at[pl.ds(j * CH, CH)], pbuf)
            pltpu.sync_copy(pbuf, acc_cf.at[cdv.at[j]], add=True)
            return carry
        lax.fori_loop(0, NCF // CH + 1, cbody, 0)

    plsc.subcore_barrier()

    @pl.when(c == 0)
    def _():
        pltpu.sync_copy(acc_cf.at[pl.ds(row0, STRIPE)],
                        cfF.at[pl.ds(row0, STRIPE)])

    @pl.when(c == 1)
    def _():
        pltpu.sync_copy(acc_cf.at[pl.ds(row0, STRIPE)],
                        cfS.at[pl.ds(row0, STRIPE)])

    for idx_hbm, nch, out in ((idxF, DFP // CH // NW, degF),
                              (idxS, DSP // CH // NW, degS)):
        pltpu.sync_copy(z16.at[pl.ds(row0, STRIPE)],
                        acc_deg.at[pl.ds(row0, STRIPE)])
        plsc.subcore_barrier()
        pltpu.sync_copy(idx_hbm.at[pl.ds(wid * nch, nch)],
                        idxb.at[pl.ds(0, nch)])

        def body(j, carry):
            pltpu.sync_copy(ones, acc_deg.at[idxb.at[j]], add=True)
            return carry
        lax.fori_loop(0, nch, body, 0)
        plsc.subcore_barrier()
        pltpu.sync_copy(acc_deg.at[pl.ds(row0, STRIPE)],
                        out.at[c].at[pl.ds(row0, STRIPE)])
        plsc.subcore_barrier()


def _sc_adj_body(yAF, yBF, yAS, yBS, sF, dF, sS, dS, z128,
                 aAF, aBF, aAS, aBS,
                 acc, sidx, didx, rowsA, rowsB, semA, semB):
    c = lax.axis_index("c")
    s = lax.axis_index("s")
    row0 = s * STRIPE

    def run(ys, si, di, ept, out):
        pltpu.sync_copy(z128.at[pl.ds(row0, STRIPE)],
                        acc.at[pl.ds(row0, STRIPE)])
        plsc.subcore_barrier()
        base_row = s * (ept // GCH)

        def outer(g, carry):
            pltpu.sync_copy(si.at[pl.ds(base_row + g * IBLK, IBLK)], sidx)
            pltpu.sync_copy(di.at[pl.ds(base_row + g * IBLK, IBLK)], didx)
            pltpu.async_copy(ys.at[sidx.at[0]], rowsA, semA)

            def body(k, carry2):
                pltpu.async_copy(ys.at[didx.at[k]], rowsB, semB)
                pltpu.make_async_copy(ys.at[sidx.at[k]], rowsA, semA).wait()
                pltpu.sync_copy(rowsA, acc.at[didx.at[k]], add=True)
                kn = jnp.minimum(k + 1, IBLK - 1)
                pltpu.async_copy(ys.at[sidx.at[kn]], rowsA, semA)
                pltpu.make_async_copy(ys.at[didx.at[k]], rowsB, semB).wait()
                pltpu.sync_copy(rowsB, acc.at[sidx.at[k]], add=True)
                return carry2
            lax.fori_loop(0, IBLK, body, 0)
            pltpu.make_async_copy(ys.at[sidx.at[IBLK - 1]], rowsA, semA).wait()
            return carry
        lax.fori_loop(0, ept // GCH // IBLK, outer, 0)
        plsc.subcore_barrier()
        pltpu.sync_copy(acc.at[pl.ds(row0, STRIPE)],
                        out.at[pl.ds(row0, STRIPE)])
        plsc.subcore_barrier()

    @pl.when(c == 0)
    def _():
        run(yAF, sF, dF, EFP // NTILES, aAF)
        run(yAS, sS, dS, ESP // NTILES, aAS)

    @pl.when(c == 1)
    def _():
        run(yBF, sF, dF, EFP // NTILES, aBF)
        run(yBS, sS, dS, ESP // NTILES, aBS)


def _sc_epi_body(x, cd, out, idxv, buf, sem):
    c = lax.axis_index("c")
    s = lax.axis_index("s")
    wid = c * NTILES + s
    pltpu.sync_copy(cd, idxv)
    pltpu.async_copy(x.at[idxv.at[wid]], buf, sem).wait()
    pltpu.sync_copy(buf, out.at[pl.ds(wid * 32, 32)])


def _ys_halves(dis, cfb, xcols):
    pieces = []
    for k in range(C):
        pieces.append(cfb[:, k * CFD:(k + 1) * CFD])
        pieces.append(xcols[k])
    pieces.append(jnp.zeros((BLK, YW - C * YD), _f32))
    ys = dis * jnp.concatenate(pieces, axis=1)
    return ys[:, :HW], ys[:, HW:]


def _tc_pre_body(opf, idsf, cfF, cfS, dgF, dgS,
                 emb, Wf, We, Wcp, b1, W2, b2,
                 x0F, x0S, yAF, yBF, yAS, yBS, disF, disS):
    oh = (idsf[...] == lax.broadcasted_iota(jnp.int32, (BLK, VOCAB), 1))
    e = _dotx(oh.astype(_f32), emb[...])
    nfb = _dot(opf[...], Wf[...]) + _dot(e, We[...]) + b1[...]
    for cf, dg, x0o, yAo, yBo, diso in ((cfF, dgF, x0F, yAF, yBF, disF),
                                        (cfS, dgS, x0S, yAS, yBS, disS)):
        dgv = dg[...]
        dis = lax.rsqrt(1.0 + dgv[0, :, 0:1] + dgv[1, :, 0:1])
        diso[...] = dis
        cfb = cf[...] * 100.0
        xcols = []
        for k in range(C):
            cfc = cfb[:, k * CFD:(k + 1) * CFD]
            t = _leaky(_dot(cfc, Wcp[...]) + nfb)
            x0c = _leaky(_dot(t, W2[...]) + b2[...])
            xcols.append(x0c)
            x0o[:, k * HID:(k + 1) * HID] = x0c
        yA, yB = _ys_halves(dis, cfb, xcols)
        yAo[...] = yA
        yBo[...] = yB


def _tc_mid1_body(x0F, x0S, yAF, yBF, yAS, yBS, pAF, pBF, pAS, pBS,
                  cfF, cfS, disF, disS,
                  W1, b1, W2, b2,
                  x1F, x1S, y2AF, y2BF, y2AS, y2BS):
    for x0, yA, yB, pA, pB, cf, dis, x1o, y2Ao, y2Bo in (
            (x0F, yAF, yBF, pAF, pBF, cfF, disF, x1F, y2AF, y2BF),
            (x0S, yAS, yBS, pAS, pBS, cfS, disS, x1S, y2AS, y2BS)):
        d = dis[...]
        a = d * jnp.concatenate([yA[...] + pA[...], yB[...] + pB[...]], axis=1)
        cfb = cf[...] * 100.0
        x0v = x0[...]
        xcols = []
        for k in range(C):
            ac = a[:, k * YD:(k + 1) * YD]
            h = _dot(_leaky(_dot(ac, W1[...]) + b1[...]), W2[...]) + b2[...]
            x1c = x0v[:, k * HID:(k + 1) * HID] + _leaky(h)
            xcols.append(x1c)
            x1o[:, k * HID:(k + 1) * HID] = x1c
        yA2, yB2 = _ys_halves(d, cfb, xcols)
        y2Ao[...] = yA2
        y2Bo[...] = yB2


def _tc_mid2_body(x1F, x1S, yAF, yBF, yAS, yBS, pAF, pBF, pAS, pBS,
                  disF, disS, self_f, gof,
                  W1, b1, W2, b2,
                  x, pooled, counts):
    xs = []
    for x1, yA, yB, pA, pB, dis in (
            (x1F, yAF, yBF, pAF, pBF, disF),
            (x1S, yAS, yBS, pAS, pBS, disS)):
        d = dis[...]
        a = d * jnp.concatenate([yA[...] + pA[...], yB[...] + pB[...]], axis=1)
        x1v = x1[...]
        cols = []
        for k in range(C):
            ac = a[:, k * YD:(k + 1) * YD]
            h = _dot(_leaky(_dot(ac, W1[...]) + b1[...]), W2[...]) + b2[...]
            cols.append(x1v[:, k * HID:(k + 1) * HID] + _leaky(h))
        xs.append(jnp.concatenate(cols, axis=1))
    sel = self_f[...]
    xb = sel * xs[1] + (1.0 - sel) * xs[0]
    x[...] = xb
    oh = (gof[...] == lax.broadcasted_iota(jnp.int32, (BLK, G), 1)).astype(_f32)
    ps = _dotx_t(oh, xb)
    cnt = _dotx_t(oh, jnp.ones((BLK, 1), _f32))

    @pl.when(pl.program_id(0) == 0)
    def _():
        pooled[...] = ps
        counts[...] = cnt

    @pl.when(pl.program_id(0) != 0)
    def _():
        pooled[...] += ps
        counts[...] += cnt


def _tc_final_body(pooled, counts, cfgx, gidc, pW1, pW2, out):
    oh = (gidc[...] == lax.broadcasted_iota(jnp.int32, (1024, G), 1)).astype(_f32)
    pc = _dotx_t(oh, cfgx[...])
    ps = pooled[...]
    pm = ps / jnp.maximum(counts[...], 1.0)
    cols = []
    for k in range(C):
        sl = slice(k * HID, (k + 1) * HID)
        h = jnp.concatenate([pm[:, sl], _l2n(ps[:, sl]), _l2n(pc[:, sl])],
                            axis=1)
        cols.append(_dot(_leaky(_dot(h, pW1[...])), pW2[...]))
    cols.append(jnp.zeros((G, 128 - C), _f32))
    out[...] = jnp.concatenate(cols, axis=1)


def kernel(op_feats, nconfig_feats, emb, pre_W1, pre_b1, pre_W2, pre_b2,
           gc1_W1, gc1_b1, gc1_W2, gc1_b2, gc2_W1, gc2_b1, gc2_W2, gc2_b2,
           post_W1, post_W2, op_ids, selected, feed_edges, sampled_feed_edges,
           config_dst, sampled_config_dst, graph_id_op, graph_id_config):
    i32 = jnp.int32
    mesh = plsc.VectorSubcoreMesh(core_axis_name="c", subcore_axis_name="s")
    scparams = pltpu.CompilerParams(use_tc_tiling_on_sc=False)

    def padidx(n):
        return N + (jnp.arange(n, dtype=i32) % 16)

    fe = feed_edges.astype(i32)
    se = sampled_feed_edges.astype(i32)
    sF = jnp.concatenate([fe[0], padidx(EFP - EF)]).reshape(EFP // GCH, GCH)
    dF = jnp.concatenate([fe[1], padidx(EFP - EF)]).reshape(EFP // GCH, GCH)
    sS = jnp.concatenate([se[0], padidx(ESP - ES)]).reshape(ESP // GCH, GCH)
    dS = jnp.concatenate([se[1], padidx(ESP - ES)]).reshape(ESP // GCH, GCH)
    idxF = jnp.concatenate([fe[0], fe[1], padidx(DFP - 2 * EF)]
                           ).reshape(DFP // CH, CH)
    idxS = jnp.concatenate([se[0], se[1], padidx(DSP - 2 * ES)]
                           ).reshape(DSP // CH, CH)
    cdF = jnp.concatenate([config_dst.astype(i32), padidx(24)]).reshape(8, CH)
    cdS = jnp.concatenate([sampled_config_dst.astype(i32), padidx(24)]
                          ).reshape(8, CH)
    cdpad = jnp.pad(config_dst.astype(i32), (0, 24)).reshape(32, 32)
    P = jnp.pad(jnp.reshape(nconfig_feats, (NCF, C * CFD)),
                ((0, 24), (0, CFW - C * CFD)))
    z128 = jnp.zeros((NP, HW), _f32)
    z96 = jnp.zeros((NP, CFW), _f32)
    z16 = jnp.zeros((NP, 16), _f32)
    idsf = op_ids.astype(i32)[:, None]
    self_f = selected.astype(_f32)[:, None]
    gof = graph_id_op.astype(i32)[:, None]
    gic = jnp.pad(graph_id_config.astype(i32), (0, 24),
                  constant_values=G)[:, None]
    Wcp, Wf, We = pre_W1[:CFD], pre_W1[CFD:CFD + OPF], pre_W1[CFD + OPF:]
    b_pre1 = pre_b1[None, :]
    b_pre2 = pre_b2[None, :]
    b_g11 = gc1_b1[None, :]
    b_g12 = gc1_b2[None, :]
    b_g21 = gc2_b1[None, :]
    b_g22 = gc2_b2[None, :]

    sds = jax.ShapeDtypeStruct

    sc_pro = pl.kernel(
        _sc_pro_body,
        out_type=[sds((NP, CFW), _f32), sds((NP, CFW), _f32),
                  sds((2, NP, 16), _f32), sds((2, NP, 16), _f32)],
        mesh=mesh,
        scratch_types=[pltpu.VMEM_SHARED((NP, CFW), _f32),
                       pltpu.VMEM_SHARED((NP, 16), _f32),
                       pltpu.VMEM((CH, CFW), _f32),
                       pltpu.VMEM((CH, 16), _f32),
                       pltpu.VMEM((DFP // CH // NW, CH), i32),
                       pltpu.VMEM((8, CH), i32)],
        compiler_params=scparams)
    cfF, cfS, degF, degS = sc_pro(idxF, idxS, cdF, cdS, P, z96, z16)

    def fullspec(shp):
        return pl.BlockSpec(shp, lambda i: tuple(0 for _ in shp))

    def blkspec(r):
        return pl.BlockSpec((BLK, r), lambda i: (i, 0))

    degspec = pl.BlockSpec((2, BLK, 16), lambda i: (0, i, 0))

    x0F, x0S, y1AF, y1BF, y1AS, y1BS, disF, disS = pl.pallas_call(
        _tc_pre_body,
        grid=(NBLK,),
        in_specs=[blkspec(OPF), blkspec(1), blkspec(CFW), blkspec(CFW),
                  degspec, degspec,
                  fullspec((VOCAB, HID)), fullspec((OPF, HID)),
                  fullspec((HID, HID)), fullspec((CFD, HID)),
                  fullspec((1, HID)), fullspec((HID, HID)),
                  fullspec((1, HID))],
        out_specs=[blkspec(RW), blkspec(RW),
                   blkspec(HW), blkspec(HW), blkspec(HW), blkspec(HW),
                   blkspec(1), blkspec(1)],
        out_shape=[sds((NP, RW), _f32), sds((NP, RW), _f32),
                   sds((NP, HW), _f32), sds((NP, HW), _f32),
                   sds((NP, HW), _f32), sds((NP, HW), _f32),
                   sds((NP, 1), _f32), sds((NP, 1), _f32)],
    )(op_feats, idsf, cfF, cfS, degF, degS,
      emb, Wf, We, Wcp, b_pre1, pre_W2, b_pre2)

    adj = pl.kernel(
        _sc_adj_body,
        out_type=[sds((NP, HW), _f32), sds((NP, HW), _f32),
                  sds((NP, HW), _f32), sds((NP, HW), _f32)],
        mesh=mesh,
        scratch_types=[pltpu.VMEM_SHARED((NP, HW), _f32),
                       pltpu.VMEM((IBLK, GCH), i32),
                       pltpu.VMEM((IBLK, GCH), i32),
                       pltpu.VMEM((GCH, HW), _f32),
                       pltpu.VMEM((GCH, HW), _f32),
                       pltpu.SemaphoreType.DMA,
                       pltpu.SemaphoreType.DMA],
        compiler_params=scparams)
    p1AF, p1BF, p1AS, p1BS = adj(y1AF, y1BF, y1AS, y1BS, sF, dF, sS, dS, z128)

    x1F, x1S, y2AF, y2BF, y2AS, y2BS = pl.pallas_call(
        _tc_mid1_body,
        grid=(NBLK,),
        in_specs=[blkspec(RW), blkspec(RW),
                  blkspec(HW), blkspec(HW), blkspec(HW), blkspec(HW),
                  blkspec(HW), blkspec(HW), blkspec(HW), blkspec(HW),
                  blkspec(CFW), blkspec(CFW), blkspec(1), blkspec(1),
                  fullspec((YD, HID)), fullspec((1, HID)),
                  fullspec((HID, HID)), fullspec((1, HID))],
        out_specs=[blkspec(RW), blkspec(RW),
                   blkspec(HW), blkspec(HW), blkspec(HW), blkspec(HW)],
        out_shape=[sds((NP, RW), _f32), sds((NP, RW), _f32),
                   sds((NP, HW), _f32), sds((NP, HW), _f32),
                   sds((NP, HW), _f32), sds((NP, HW), _f32)],
    )(x0F, x0S, y1AF, y1BF, y1AS, y1BS, p1AF, p1BF, p1AS, p1BS,
      cfF, cfS, disF, disS,
      gc1_W1, b_g11, gc1_W2, b_g12)

    p2AF, p2BF, p2AS, p2BS = adj(y2AF, y2BF, y2AS, y2BS, sF, dF, sS, dS, z128)

    x, pooled, counts = pl.pallas_call(
        _tc_mid2_body,
        grid=(NBLK,),
        in_specs=[blkspec(RW), blkspec(RW),
                  blkspec(HW), blkspec(HW), blkspec(HW), blkspec(HW),
                  blkspec(HW), blkspec(HW), blkspec(HW), blkspec(HW),
                  blkspec(1), blkspec(1), blkspec(1), blkspec(1),
                  fullspec((YD, HID)), fullspec((1, HID)),
                  fullspec((HID, HID)), fullspec((1, HID))],
        out_specs=[blkspec(RW),
                   pl.BlockSpec((G, RW), lambda i: (0, 0)),
                   pl.BlockSpec((G, 1), lambda i: (0, 0))],
        out_shape=[sds((NP, RW), _f32), sds((G, RW), _f32), sds((G, 1), _f32)],
    )(x1F, x1S, y2AF, y2BF, y2AS, y2BS, p2AF, p2BF, p2AS, p2BS,
      disF, disS, self_f, gof,
      gc2_W1, b_g21, gc2_W2, b_g22)

    sc_epi = pl.kernel(
        _sc_epi_body,
        out_type=sds((1024, RW), _f32),
        mesh=mesh,
        scratch_types=[pltpu.VMEM((32, 32), i32),
                       pltpu.VMEM((32, RW), _f32),
                       pltpu.SemaphoreType.DMA],
        compiler_params=scparams)
    cfgx = sc_epi(x, cdpad)

    out8 = pl.pallas_call(
        _tc_final_body,
        grid=(1,),
        in_specs=[fullspec((G, RW)), fullspec((G, 1)),
                  fullspec((1024, RW)), fullspec((1024, 1)),
                  fullspec((3 * HID, HID)), fullspec((HID, 1))],
        out_specs=[fullspec((G, 128))],
        out_shape=[sds((G, 128), _f32)],
    )(pooled, counts, cfgx, gic, post_W1, post_W2)[0]

    return out8[:, :C]


# speedup vs baseline: 155.4393x; 1.0710x over previous
import jax
import jax.numpy as jnp
from jax import lax
from jax.experimental import pallas as pl
from jax.experimental.pallas import tpu as pltpu
from jax.experimental.pallas import tpu_sc as plsc

N = 10000
C = 5
HID = 32
RW = C * HID
YD = 50
YW = 256
HW = 128
CFD = 18
CFW = 96
G = 8
VOCAB = 120
OPF = 140
EF = 320000
ES = 160000
NCF = 1000
CH = 128
NW = 32
EFP = 327680
ESP = 163840
DFP = 655360
DSP = 327680
NP = 10112
NTILES = 16
STRIPE = NP // NTILES
ALPHA = 0.2
BLK = 400
NBLK = N // BLK
GCH = 128
IBLK = 16

_f32 = jnp.float32


def _leaky(x):
    return jnp.where(x > 0, x, ALPHA * x)


def _dot(a, b):
    return jnp.dot(a, b, precision=lax.Precision.DEFAULT,
                   preferred_element_type=_f32)


def _dotx(a, b):
    return jnp.dot(a, b, precision=lax.Precision.HIGHEST,
                   preferred_element_type=_f32)


def _dotx_t(a, b):
    return lax.dot_general(a, b, (((0,), (0,)), ((), ())),
                           precision=lax.Precision.HIGHEST,
                           preferred_element_type=_f32)


def _l2n(y):
    return y * lax.rsqrt(jnp.maximum(jnp.sum(y * y, axis=-1, keepdims=True),
                                     1e-12))


def _sc_pro_body(idxF, idxS, cdF, cdS, P, z96, z16,
                 cfF, cfS, degF, degS,
                 acc_cf, acc_deg, pbuf, ones, idxb, cdv):
    c = lax.axis_index("c")
    s = lax.axis_index("s")
    wid = c * NTILES + s
    row0 = s * STRIPE

    def fill(j, carry):
        ones[j] = jnp.ones((16,), _f32)
        return carry
    lax.fori_loop(0, CH, fill, 0)

    pltpu.sync_copy(z96.at[pl.ds(row0, STRIPE)], acc_cf.at[pl.ds(row0, STRIPE)])
    plsc.subcore_barrier()

    @pl.when((c == 0) & (s == 0))
    def _():
        pltpu.sync_copy(cdF, cdv)

    @pl.when((c == 1) & (s == 0))
    def _():
        pltpu.sync_copy(cdS, cdv)

    @pl.when(s == 0)
    def _():
        def cbody(j, carry):
            pltpu.sync_copy(P.at[pl.ds(j * CH, CH)], pbuf)
            pltpu.sync_copy(pbuf, acc_cf.at[cdv.at[j]], add=True)
            return carry
        lax.fori_loop(0, NCF // CH + 1, cbody, 0)

    plsc.subcore_barrier()

    @pl.when(c == 0)
    def _():
        pltpu.sync_copy(acc_cf.at[pl.ds(row0, STRIPE)],
                        cfF.at[pl.ds(row0, STRIPE)])

    @pl.when(c == 1)
    def _():
        pltpu.sync_copy(acc_cf.at[pl.ds(row0, STRIPE)],
                        cfS.at[pl.ds(row0, STRIPE)])

    for idx_hbm, nch, out in ((idxF, DFP // CH // NW, degF),
                              (idxS, DSP // CH // NW, degS)):
        pltpu.sync_copy(z16.at[pl.ds(row0, STRIPE)],
                        acc_deg.at[pl.ds(row0, STRIPE)])
        plsc.subcore_barrier()
        pltpu.sync_copy(idx_hbm.at[pl.ds(wid * nch, nch)],
                        idxb.at[pl.ds(0, nch)])

        def body(j, carry):
            pltpu.sync_copy(ones, acc_deg.at[idxb.at[j]], add=True)
            return carry
        lax.fori_loop(0, nch, body, 0)
        plsc.subcore_barrier()
        pltpu.sync_copy(acc_deg.at[pl.ds(row0, STRIPE)],
                        out.at[c].at[pl.ds(row0, STRIPE)])
        plsc.subcore_barrier()


def _sc_adj_body(yAF, yBF, yAS, yBS, sF, dF, sS, dS, z128,
                 aAF, aBF, aAS, aBS,
                 acc, sidx, didx, rowsA, rowsB, semA, semB):
    c = lax.axis_index("c")
    s = lax.axis_index("s")
    row0 = s * STRIPE

    def run(ys, si, di, ept, out):
        pltpu.sync_copy(z128.at[pl.ds(row0, STRIPE)],
                        acc.at[pl.ds(row0, STRIPE)])
        plsc.subcore_barrier()
        base_row = s * (ept // GCH)

        def outer(g, carry):
            pltpu.sync_copy(si.at[pl.ds(base_row + g * IBLK, IBLK)], sidx)
            pltpu.sync_copy(di.at[pl.ds(base_row + g * IBLK, IBLK)], didx)
            pltpu.async_copy(ys.at[sidx.at[0]], rowsA, semA)

            def body(k, carry2):
                pltpu.async_copy(ys.at[didx.at[k]], rowsB, semB)
                pltpu.make_async_copy(ys.at[sidx.at[k]], rowsA, semA).wait()
                pltpu.sync_copy(rowsA, acc.at[didx.at[k]], add=True)
                kn = jnp.minimum(k + 1, IBLK - 1)
                pltpu.async_copy(ys.at[sidx.at[kn]], rowsA, semA)
                pltpu.make_async_copy(ys.at[didx.at[k]], rowsB, semB).wait()
                pltpu.sync_copy(rowsB, acc.at[sidx.at[k]], add=True)
                return carry2
            lax.fori_loop(0, IBLK, body, 0)
            pltpu.make_async_copy(ys.at[sidx.at[IBLK - 1]], rowsA, semA).wait()
            return carry
        lax.fori_loop(0, ept // GCH // IBLK, outer, 0)
        plsc.subcore_barrier()
        pltpu.sync_copy(acc.at[pl.ds(row0, STRIPE)],
                        out.at[pl.ds(row0, STRIPE)])
        plsc.subcore_barrier()

    @pl.when(c == 0)
    def _():
        run(yAF, sF, dF, EFP // NTILES, aAF)
        run(yAS, sS, dS, ESP // NTILES, aAS)

    @pl.when(c == 1)
    def _():
        run(yBF, sF, dF, EFP // NTILES, aBF)
        run(yBS, sS, dS, ESP // NTILES, aBS)


def _sc_epi_body(x, cd, out, idxv, buf, sem):
    c = lax.axis_index("c")
    s = lax.axis_index("s")
    wid = c * NTILES + s
    pltpu.sync_copy(cd, idxv)
    pltpu.async_copy(x.at[idxv.at[wid]], buf, sem).wait()
    pltpu.sync_copy(buf, out.at[pl.ds(wid * 32, 32)])


def _ys_halves(dis, cfb, xcols):
    ys = dis * jnp.concatenate([cfb] + xcols, axis=1)
    return ys[:, :HW], ys[:, HW:]


def _a_cfg(acf, ax, k):
    return jnp.concatenate([acf[:, k * CFD:(k + 1) * CFD],
                            ax[:, k * HID:(k + 1) * HID]], axis=1)


def _tc_pre_body(opf, idsf, cfF, cfS, dgF, dgS,
                 emb, Wf, We, Wcp, b1, W2, b2,
                 x0F, x0S, yAF, yBF, yAS, yBS, disF, disS):
    oh = (idsf[...] == lax.broadcasted_iota(jnp.int32, (BLK, VOCAB), 1))
    e = _dotx(oh.astype(_f32), emb[...])
    nfb = _dot(opf[...], Wf[...]) + _dot(e, We[...]) + b1[...]
    for cf, dg, x0o, yAo, yBo, diso in ((cfF, dgF, x0F, yAF, yBF, disF),
                                        (cfS, dgS, x0S, yAS, yBS, disS)):
        dgv = dg[...]
        dis = lax.rsqrt(1.0 + dgv[0, :, 0:1] + dgv[1, :, 0:1])
        diso[...] = dis
        cfb = cf[...] * 100.0
        xcols = []
        for k in range(C):
            cfc = cfb[:, k * CFD:(k + 1) * CFD]
            t = _leaky(_dot(cfc, Wcp[...]) + nfb)
            x0c = _leaky(_dot(t, W2[...]) + b2[...])
            xcols.append(x0c)
            x0o[:, k * HID:(k + 1) * HID] = x0c
        yA, yB = _ys_halves(dis, cfb, xcols)
        yAo[...] = yA
        yBo[...] = yB


def _tc_mid1_body(x0F, x0S, yAF, yBF, yAS, yBS, pAF, pBF, pAS, pBS,
                  disF, disS,
                  W1, b1, W2, b2,
                  x1F, x1S, y2AF, y2BF, y2AS, y2BS):
    for x0, yA, yB, pA, pB, dis, x1o, y2Ao, y2Bo in (
            (x0F, yAF, yBF, pAF, pBF, disF, x1F, y2AF, y2BF),
            (x0S, yAS, yBS, pAS, pBS, disS, x1S, y2AS, y2BS)):
        d = dis[...]
        full = jnp.concatenate([yA[...] + pA[...], yB[...] + pB[...]], axis=1)
        acf = d * full[:, :CFW]
        ax = d * full[:, CFW:]
        x0v = x0[...]
        xcols = []
        for k in range(C):
            ac = _a_cfg(acf, ax, k)
            h = _dot(_leaky(_dot(ac, W1[...]) + b1[...]), W2[...]) + b2[...]
            x1c = x0v[:, k * HID:(k + 1) * HID] + _leaky(h)
            xcols.append(x1c)
            x1o[:, k * HID:(k + 1) * HID] = x1c
        yx = d * jnp.concatenate(xcols, axis=1)
        y2Ao[...] = yx[:, :RW // 2]
        y2Bo[...] = yx[:, RW // 2:]


def _tc_mid2_body(x1F, x1S, y1AF, y1AS, p1AF, p1AS,
                  y2AF, y2BF, y2AS, y2BS, pAF, pBF, pAS, pBS,
                  disF, disS, self_f, gof,
                  W1, b1, W2, b2,
                  x, pooled, counts):
    xs = []
    for x1, y1A, p1A, y2A, y2B, pA, pB, dis in (
            (x1F, y1AF, p1AF, y2AF, y2BF, pAF, pBF, disF),
            (x1S, y1AS, p1AS, y2AS, y2BS, pAS, pBS, disS)):
        d = dis[...]
        acf = d * (y1A[...][:, :CFW] + p1A[...][:, :CFW])
        ax = d * jnp.concatenate([y2A[...] + pA[...], y2B[...] + pB[...]],
                                 axis=1)
        x1v = x1[...]
        cols = []
        for k in range(C):
            ac = _a_cfg(acf, ax, k)
            h = _dot(_leaky(_dot(ac, W1[...]) + b1[...]), W2[...]) + b2[...]
            cols.append(x1v[:, k * HID:(k + 1) * HID] + _leaky(h))
        xs.append(jnp.concatenate(cols, axis=1))
    sel = self_f[...]
    xb = sel * xs[1] + (1.0 - sel) * xs[0]
    x[...] = xb
    oh = (gof[...] == lax.broadcasted_iota(jnp.int32, (BLK, G), 1)).astype(_f32)
    ps = _dotx_t(oh, xb)
    cnt = _dotx_t(oh, jnp.ones((BLK, 1), _f32))

    @pl.when(pl.program_id(0) == 0)
    def _():
        pooled[...] = ps
        counts[...] = cnt

    @pl.when(pl.program_id(0) != 0)
    def _():
        pooled[...] += ps
        counts[...] += cnt


def _tc_final_body(pooled, counts, cfgx, gidc, pW1, pW2, out):
    oh = (gidc[...] == lax.broadcasted_iota(jnp.int32, (1024, G), 1)).astype(_f32)
    pc = _dotx_t(oh, cfgx[...])
    ps = pooled[...]
    pm = ps / jnp.maximum(counts[...], 1.0)
    cols = []
    for k in range(C):
        sl = slice(k * HID, (k + 1) * HID)
        h = jnp.concatenate([pm[:, sl], _l2n(ps[:, sl]), _l2n(pc[:, sl])],
                            axis=1)
        cols.append(_dot(_leaky(_dot(h, pW1[...])), pW2[...]))
    cols.append(jnp.zeros((G, 128 - C), _f32))
    out[...] = jnp.concatenate(cols, axis=1)


def kernel(op_feats, nconfig_feats, emb, pre_W1, pre_b1, pre_W2, pre_b2,
           gc1_W1, gc1_b1, gc1_W2, gc1_b2, gc2_W1, gc2_b1, gc2_W2, gc2_b2,
           post_W1, post_W2, op_ids, selected, feed_edges, sampled_feed_edges,
           config_dst, sampled_config_dst, graph_id_op, graph_id_config):
    i32 = jnp.int32
    mesh = plsc.VectorSubcoreMesh(core_axis_name="c", subcore_axis_name="s")
    scparams = pltpu.CompilerParams(use_tc_tiling_on_sc=False)

    def padidx(n):
        return N + (jnp.arange(n, dtype=i32) % 16)

    fe = feed_edges.astype(i32)
    se = sampled_feed_edges.astype(i32)
    sF = jnp.concatenate([fe[0], padidx(EFP - EF)]).reshape(EFP // GCH, GCH)
    dF = jnp.concatenate([fe[1], padidx(EFP - EF)]).reshape(EFP // GCH, GCH)
    sS = jnp.concatenate([se[0], padidx(ESP - ES)]).reshape(ESP // GCH, GCH)
    dS = jnp.concatenate([se[1], padidx(ESP - ES)]).reshape(ESP // GCH, GCH)
    idxF = jnp.concatenate([fe[0], fe[1], padidx(DFP - 2 * EF)]
                           ).reshape(DFP // CH, CH)
    idxS = jnp.concatenate([se[0], se[1], padidx(DSP - 2 * ES)]
                           ).reshape(DSP // CH, CH)
    cdF = jnp.concatenate([config_dst.astype(i32), padidx(24)]).reshape(8, CH)
    cdS = jnp.concatenate([sampled_config_dst.astype(i32), padidx(24)]
                          ).reshape(8, CH)
    cdpad = jnp.pad(config_dst.astype(i32), (0, 24)).reshape(32, 32)
    P = jnp.pad(jnp.reshape(nconfig_feats, (NCF, C * CFD)),
                ((0, 24), (0, CFW - C * CFD)))
    z128 = jnp.zeros((NP, HW), _f32)
    z80 = jnp.zeros((NP, RW // 2), _f32)
    z96 = jnp.zeros((NP, CFW), _f32)
    z16 = jnp.zeros((NP, 16), _f32)
    idsf = op_ids.astype(i32)[:, None]
    self_f = selected.astype(_f32)[:, None]
    gof = graph_id_op.astype(i32)[:, None]
    gic = jnp.pad(graph_id_config.astype(i32), (0, 24),
                  constant_values=G)[:, None]
    Wcp, Wf, We = pre_W1[:CFD], pre_W1[CFD:CFD + OPF], pre_W1[CFD + OPF:]
    b_pre1 = pre_b1[None, :]
    b_pre2 = pre_b2[None, :]
    b_g11 = gc1_b1[None, :]
    b_g12 = gc1_b2[None, :]
    b_g21 = gc2_b1[None, :]
    b_g22 = gc2_b2[None, :]

    sds = jax.ShapeDtypeStruct

    sc_pro = pl.kernel(
        _sc_pro_body,
        out_type=[sds((NP, CFW), _f32), sds((NP, CFW), _f32),
                  sds((2, NP, 16), _f32), sds((2, NP, 16), _f32)],
        mesh=mesh,
        scratch_types=[pltpu.VMEM_SHARED((NP, CFW), _f32),
                       pltpu.VMEM_SHARED((NP, 16), _f32),
                       pltpu.VMEM((CH, CFW), _f32),
                       pltpu.VMEM((CH, 16), _f32),
                       pltpu.VMEM((DFP // CH // NW, CH), i32),
                       pltpu.VMEM((8, CH), i32)],
        compiler_params=scparams)
    cfF, cfS, degF, degS = sc_pro(idxF, idxS, cdF, cdS, P, z96, z16)

    def fullspec(shp):
        return pl.BlockSpec(shp, lambda i: tuple(0 for _ in shp))

    def blkspec(r):
        return pl.BlockSpec((BLK, r), lambda i: (i, 0))

    degspec = pl.BlockSpec((2, BLK, 16), lambda i: (0, i, 0))

    x0F, x0S, y1AF, y1BF, y1AS, y1BS, disF, disS = pl.pallas_call(
        _tc_pre_body,
        grid=(NBLK,),
        in_specs=[blkspec(OPF), blkspec(1), blkspec(CFW), blkspec(CFW),
                  degspec, degspec,
                  fullspec((VOCAB, HID)), fullspec((OPF, HID)),
                  fullspec((HID, HID)), fullspec((CFD, HID)),
                  fullspec((1, HID)), fullspec((HID, HID)),
                  fullspec((1, HID))],
        out_specs=[blkspec(RW), blkspec(RW),
                   blkspec(HW), blkspec(HW), blkspec(HW), blkspec(HW),
                   blkspec(1), blkspec(1)],
        out_shape=[sds((NP, RW), _f32), sds((NP, RW), _f32),
                   sds((NP, HW), _f32), sds((NP, HW), _f32),
                   sds((NP, HW), _f32), sds((NP, HW), _f32),
                   sds((NP, 1), _f32), sds((NP, 1), _f32)],
    )(op_feats, idsf, cfF, cfS, degF, degS,
      emb, Wf, We, Wcp, b_pre1, pre_W2, b_pre2)

    adj = pl.kernel(
        _sc_adj_body,
        out_type=[sds((NP, HW), _f32), sds((NP, HW), _f32),
                  sds((NP, HW), _f32), sds((NP, HW), _f32)],
        mesh=mesh,
        scratch_types=[pltpu.VMEM_SHARED((NP, HW), _f32),
                       pltpu.VMEM((IBLK, GCH), i32),
                       pltpu.VMEM((IBLK, GCH), i32),
                       pltpu.VMEM((GCH, HW), _f32),
                       pltpu.VMEM((GCH, HW), _f32),
                       pltpu.SemaphoreType.DMA,
                       pltpu.SemaphoreType.DMA],
        compiler_params=scparams)
    p1AF, p1BF, p1AS, p1BS = adj(y1AF, y1BF, y1AS, y1BS, sF, dF, sS, dS, z128)

    HX = RW // 2
    x1F, x1S, y2AF, y2BF, y2AS, y2BS = pl.pallas_call(
        _tc_mid1_body,
        grid=(NBLK,),
        in_specs=[blkspec(RW), blkspec(RW),
                  blkspec(HW), blkspec(HW), blkspec(HW), blkspec(HW),
                  blkspec(HW), blkspec(HW), blkspec(HW), blkspec(HW),
                  blkspec(1), blkspec(1),
                  fullspec((YD, HID)), fullspec((1, HID)),
                  fullspec((HID, HID)), fullspec((1, HID))],
        out_specs=[blkspec(RW), blkspec(RW),
                   blkspec(HX), blkspec(HX), blkspec(HX), blkspec(HX)],
        out_shape=[sds((NP, RW), _f32), sds((NP, RW), _f32),
                   sds((NP, HX), _f32), sds((NP, HX), _f32),
                   sds((NP, HX), _f32), sds((NP, HX), _f32)],
    )(x0F, x0S, y1AF, y1BF, y1AS, y1BS, p1AF, p1BF, p1AS, p1BS,
      disF, disS,
      gc1_W1, b_g11, gc1_W2, b_g12)

    adj2 = pl.kernel(
        _sc_adj_body,
        out_type=[sds((NP, HX), _f32), sds((NP, HX), _f32),
                  sds((NP, HX), _f32), sds((NP, HX), _f32)],
        mesh=mesh,
        scratch_types=[pltpu.VMEM_SHARED((NP, HX), _f32),
                       pltpu.VMEM((IBLK, GCH), i32),
                       pltpu.VMEM((IBLK, GCH), i32),
                       pltpu.VMEM((GCH, HX), _f32),
                       pltpu.VMEM((GCH, HX), _f32),
                       pltpu.SemaphoreType.DMA,
                       pltpu.SemaphoreType.DMA],
        compiler_params=scparams)
    p2AF, p2BF, p2AS, p2BS = adj2(y2AF, y2BF, y2AS, y2BS, sF, dF, sS, dS, z80)

    x, pooled, counts = pl.pallas_call(
        _tc_mid2_body,
        grid=(NBLK,),
        in_specs=[blkspec(RW), blkspec(RW),
                  blkspec(HW), blkspec(HW), blkspec(HW), blkspec(HW),
                  blkspec(HX), blkspec(HX), blkspec(HX), blkspec(HX),
                  blkspec(HX), blkspec(HX), blkspec(HX), blkspec(HX),
                  blkspec(1), blkspec(1), blkspec(1), blkspec(1),
                  fullspec((YD, HID)), fullspec((1, HID)),
                  fullspec((HID, HID)), fullspec((1, HID))],
        out_specs=[blkspec(RW),
                   pl.BlockSpec((G, RW), lambda i: (0, 0)),
                   pl.BlockSpec((G, 1), lambda i: (0, 0))],
        out_shape=[sds((NP, RW), _f32), sds((G, RW), _f32), sds((G, 1), _f32)],
    )(x1F, x1S, y1AF, y1AS, p1AF, p1AS,
      y2AF, y2BF, y2AS, y2BS, p2AF, p2BF, p2AS, p2BS,
      disF, disS, self_f, gof,
      gc2_W1, b_g21, gc2_W2, b_g22)

    sc_epi = pl.kernel(
        _sc_epi_body,
        out_type=sds((1024, RW), _f32),
        mesh=mesh,
        scratch_types=[pltpu.VMEM((32, 32), i32),
                       pltpu.VMEM((32, RW), _f32),
                       pltpu.SemaphoreType.DMA],
        compiler_params=scparams)
    cfgx = sc_epi(x, cdpad)

    out8 = pl.pallas_call(
        _tc_final_body,
        grid=(1,),
        in_specs=[fullspec((G, RW)), fullspec((G, 1)),
                  fullspec((1024, RW)), fullspec((1024, 1)),
                  fullspec((3 * HID, HID)), fullspec((HID, 1))],
        out_specs=[fullspec((G, 128))],
        out_shape=[sds((G, 128), _f32)],
    )(pooled, counts, cfgx, gic, post_W1, post_W2)[0]

    return out8[:, :C]


# speedup vs baseline: 162.0397x; 1.0425x over previous
import jax
import jax.numpy as jnp
from jax import lax
from jax.experimental import pallas as pl
from jax.experimental.pallas import tpu as pltpu
from jax.experimental.pallas import tpu_sc as plsc

N = 10000
C = 5
HID = 32
RW = C * HID
YD = 50
YW = 256
HW = 128
CFD = 18
CFW = 96
G = 8
VOCAB = 120
OPF = 140
EF = 320000
ES = 160000
NCF = 1000
CH = 128
NW = 32
EFP = 327680
ESP = 163840
DFP = 655360
DSP = 327680
NP = 10112
NTILES = 16
STRIPE = NP // NTILES
ALPHA = 0.2
BLK = 400
NBLK = N // BLK
GCH = 128
IBLK = 16

_f32 = jnp.float32


def _leaky(x):
    return jnp.where(x > 0, x, ALPHA * x)


def _dot(a, b):
    return jnp.dot(a, b, precision=lax.Precision.DEFAULT,
                   preferred_element_type=_f32)


def _dotx(a, b):
    return jnp.dot(a, b, precision=lax.Precision.HIGHEST,
                   preferred_element_type=_f32)


def _dotx_t(a, b):
    return lax.dot_general(a, b, (((0,), (0,)), ((), ())),
                           precision=lax.Precision.HIGHEST,
                           preferred_element_type=_f32)


def _l2n(y):
    return y * lax.rsqrt(jnp.maximum(jnp.sum(y * y, axis=-1, keepdims=True),
                                     1e-12))


def _sc_pro_body(idxF, idxS, cdF, cdS, P, z96, z16,
                 cfF, cfS, degF, degS,
                 acc_cf, acc_deg, pbuf, ones, idxb, cdv):
    c = lax.axis_index("c")
    s = lax.axis_index("s")
    wid = c * NTILES + s
    row0 = s * STRIPE

    def fill(j, carry):
        ones[j] = jnp.ones((16,), _f32)
        return carry
    lax.fori_loop(0, CH, fill, 0)

    pltpu.sync_copy(z96.at[pl.ds(row0, STRIPE)], acc_cf.at[pl.ds(row0, STRIPE)])
    plsc.subcore_barrier()

    @pl.when((c == 0) & (s == 0))
    def _():
        pltpu.sync_copy(cdF, cdv)

    @pl.when((c == 1) & (s == 0))
    def _():
        pltpu.sync_copy(cdS, cdv)

    @pl.when(s == 0)
    def _():
        def cbody(j, carry):
            pltpu.sync_copy(P.at[pl.ds(j * CH, CH)], pbuf)
            pltpu.sync_copy(pbuf, acc_cf.at[cdv.at[j]], add=True)
            return carry
        lax.fori_loop(0, NCF // CH + 1, cbody, 0)

    plsc.subcore_barrier()

    @pl.when(c == 0)
    def _():
        pltpu.sync_copy(acc_cf.at[pl.ds(row0, STRIPE)],
                        cfF.at[pl.ds(row0, STRIPE)])

    @pl.when(c == 1)
    def _():
        pltpu.sync_copy(acc_cf.at[pl.ds(row0, STRIPE)],
                        cfS.at[pl.ds(row0, STRIPE)])

    for idx_hbm, nch, out in ((idxF, DFP // CH // NW, degF),
                              (idxS, DSP // CH // NW, degS)):
        pltpu.sync_copy(z16.at[pl.ds(row0, STRIPE)],
                        acc_deg.at[pl.ds(row0, STRIPE)])
        plsc.subcore_barrier()
        pltpu.sync_copy(idx_hbm.at[pl.ds(wid * nch, nch)],
                        idxb.at[pl.ds(0, nch)])

        def body(j, carry):
            pltpu.sync_copy(ones, acc_deg.at[idxb.at[j]], add=True)
            return carry
        lax.fori_loop(0, nch, body, 0)
        plsc.subcore_barrier()
        pltpu.sync_copy(acc_deg.at[pl.ds(row0, STRIPE)],
                        out.at[c].at[pl.ds(row0, STRIPE)])
        plsc.subcore_barrier()


def _sc_adj_body(yA, yB, si, di, zz,
                 aA, aB,
                 acc, sidx, didx, rowsA, rowsB, semA, semB):
    c = lax.axis_index("c")
    s = lax.axis_index("s")
    row0 = s * STRIPE

    def run(ys, ept, out):
        pltpu.sync_copy(zz.at[pl.ds(row0, STRIPE)],
                        acc.at[pl.ds(row0, STRIPE)])
        plsc.subcore_barrier()
        base_row = s * (ept // GCH)

        def outer(g, carry):
            pltpu.sync_copy(si.at[pl.ds(base_row + g * IBLK, IBLK)], sidx)
            pltpu.sync_copy(di.at[pl.ds(base_row + g * IBLK, IBLK)], didx)
            pltpu.async_copy(ys.at[sidx.at[0]], rowsA, semA)

            def body(k, carry2):
                pltpu.async_copy(ys.at[didx.at[k]], rowsB, semB)
                pltpu.make_async_copy(ys.at[sidx.at[k]], rowsA, semA).wait()
                pltpu.sync_copy(rowsA, acc.at[didx.at[k]], add=True)
                kn = jnp.minimum(k + 1, IBLK - 1)
                pltpu.async_copy(ys.at[sidx.at[kn]], rowsA, semA)
                pltpu.make_async_copy(ys.at[didx.at[k]], rowsB, semB).wait()
                pltpu.sync_copy(rowsB, acc.at[sidx.at[k]], add=True)
                return carry2
            lax.fori_loop(0, IBLK, body, 0)
            pltpu.make_async_copy(ys.at[sidx.at[IBLK - 1]], rowsA, semA).wait()
            return carry
        lax.fori_loop(0, ept // GCH // IBLK, outer, 0)
        plsc.subcore_barrier()
        pltpu.sync_copy(acc.at[pl.ds(row0, STRIPE)],
                        out.at[pl.ds(row0, STRIPE)])
        plsc.subcore_barrier()

    ept = si.shape[0] * GCH // NTILES

    @pl.when(c == 0)
    def _():
        run(yA, ept, aA)

    @pl.when(c == 1)
    def _():
        run(yB, ept, aB)


def _sc_epi_body(x, cd, out, idxv, buf, sem):
    c = lax.axis_index("c")
    s = lax.axis_index("s")
    wid = c * NTILES + s
    pltpu.sync_copy(cd, idxv)
    pltpu.async_copy(x.at[idxv.at[wid]], buf, sem).wait()
    pltpu.sync_copy(buf, out.at[pl.ds(wid * 32, 32)])


def _ys_halves(dis, cfb, xcols):
    ys = dis * jnp.concatenate([cfb] + xcols, axis=1)
    return ys[:, :HW], ys[:, HW:]


def _a_cfg(acf, ax, k):
    return jnp.concatenate([acf[:, k * CFD:(k + 1) * CFD],
                            ax[:, k * HID:(k + 1) * HID]], axis=1)


def _tc_pre_body(opf, idsf, cf, dg,
                 emb, Wf, We, Wcp, b1, W2, b2,
                 x0o, yAo, yBo, diso):
    oh = (idsf[...] == lax.broadcasted_iota(jnp.int32, (BLK, VOCAB), 1))
    e = _dotx(oh.astype(_f32), emb[...])
    nfb = _dot(opf[...], Wf[...]) + _dot(e, We[...]) + b1[...]
    dgv = dg[...]
    dis = lax.rsqrt(1.0 + dgv[0, :, 0:1] + dgv[1, :, 0:1])
    diso[...] = dis
    cfb = cf[...] * 100.0
    xcols = []
    for k in range(C):
        cfc = cfb[:, k * CFD:(k + 1) * CFD]
        t = _leaky(_dot(cfc, Wcp[...]) + nfb)
        x0c = _leaky(_dot(t, W2[...]) + b2[...])
        xcols.append(x0c)
        x0o[:, k * HID:(k + 1) * HID] = x0c
    yA, yB = _ys_halves(dis, cfb, xcols)
    yAo[...] = yA
    yBo[...] = yB


def _tc_mid1_body(x0, yA, yB, pA, pB, dis,
                  W1, b1, W2, b2,
                  x1o, y2Ao, y2Bo):
    d = dis[...]
    full = jnp.concatenate([yA[...] + pA[...], yB[...] + pB[...]], axis=1)
    acf = d * full[:, :CFW]
    ax = d * full[:, CFW:]
    x0v = x0[...]
    xcols = []
    for k in range(C):
        ac = _a_cfg(acf, ax, k)
        h = _dot(_leaky(_dot(ac, W1[...]) + b1[...]), W2[...]) + b2[...]
        x1c = x0v[:, k * HID:(k + 1) * HID] + _leaky(h)
        xcols.append(x1c)
        x1o[:, k * HID:(k + 1) * HID] = x1c
    yx = d * jnp.concatenate(xcols, axis=1)
    y2Ao[...] = yx[:, :RW // 2]
    y2Bo[...] = yx[:, RW // 2:]


def _tc_mid2_body(x1F, x1S, y1AF, y1AS, p1AF, p1AS,
                  y2AF, y2BF, y2AS, y2BS, pAF, pBF, pAS, pBS,
                  disF, disS, self_f, gof,
                  W1, b1, W2, b2,
                  x, pooled, counts):
    xs = []
    for x1, y1A, p1A, y2A, y2B, pA, pB, dis in (
            (x1F, y1AF, p1AF, y2AF, y2BF, pAF, pBF, disF),
            (x1S, y1AS, p1AS, y2AS, y2BS, pAS, pBS, disS)):
        d = dis[...]
        acf = d * (y1A[...][:, :CFW] + p1A[...][:, :CFW])
        ax = d * jnp.concatenate([y2A[...] + pA[...], y2B[...] + pB[...]],
                                 axis=1)
        x1v = x1[...]
        cols = []
        for k in range(C):
            ac = _a_cfg(acf, ax, k)
            h = _dot(_leaky(_dot(ac, W1[...]) + b1[...]), W2[...]) + b2[...]
            cols.append(x1v[:, k * HID:(k + 1) * HID] + _leaky(h))
        xs.append(jnp.concatenate(cols, axis=1))
    sel = self_f[...]
    xb = sel * xs[1] + (1.0 - sel) * xs[0]
    x[...] = xb
    oh = (gof[...] == lax.broadcasted_iota(jnp.int32, (BLK, G), 1)).astype(_f32)
    ps = _dotx_t(oh, xb)
    cnt = _dotx_t(oh, jnp.ones((BLK, 1), _f32))

    @pl.when(pl.program_id(0) == 0)
    def _():
        pooled[...] = ps
        counts[...] = cnt

    @pl.when(pl.program_id(0) != 0)
    def _():
        pooled[...] += ps
        counts[...] += cnt


def _tc_final_body(pooled, counts, cfgx, gidc, pW1, pW2, out):
    oh = (gidc[...] == lax.broadcasted_iota(jnp.int32, (1024, G), 1)).astype(_f32)
    pc = _dotx_t(oh, cfgx[...])
    ps = pooled[...]
    pm = ps / jnp.maximum(counts[...], 1.0)
    cols = []
    for k in range(C):
        sl = slice(k * HID, (k + 1) * HID)
        h = jnp.concatenate([pm[:, sl], _l2n(ps[:, sl]), _l2n(pc[:, sl])],
                            axis=1)
        cols.append(_dot(_leaky(_dot(h, pW1[...])), pW2[...]))
    cols.append(jnp.zeros((G, 128 - C), _f32))
    out[...] = jnp.concatenate(cols, axis=1)


def kernel(op_feats, nconfig_feats, emb, pre_W1, pre_b1, pre_W2, pre_b2,
           gc1_W1, gc1_b1, gc1_W2, gc1_b2, gc2_W1, gc2_b1, gc2_W2, gc2_b2,
           post_W1, post_W2, op_ids, selected, feed_edges, sampled_feed_edges,
           config_dst, sampled_config_dst, graph_id_op, graph_id_config):
    i32 = jnp.int32
    mesh = plsc.VectorSubcoreMesh(core_axis_name="c", subcore_axis_name="s")
    scparams = pltpu.CompilerParams(use_tc_tiling_on_sc=False)

    def padidx(n):
        return N + (jnp.arange(n, dtype=i32) % 16)

    fe = feed_edges.astype(i32)
    se = sampled_feed_edges.astype(i32)
    sF = jnp.concatenate([fe[0], padidx(EFP - EF)]).reshape(EFP // GCH, GCH)
    dF = jnp.concatenate([fe[1], padidx(EFP - EF)]).reshape(EFP // GCH, GCH)
    sS = jnp.concatenate([se[0], padidx(ESP - ES)]).reshape(ESP // GCH, GCH)
    dS = jnp.concatenate([se[1], padidx(ESP - ES)]).reshape(ESP // GCH, GCH)
    idxF = jnp.concatenate([fe[0], fe[1], padidx(DFP - 2 * EF)]
                           ).reshape(DFP // CH, CH)
    idxS = jnp.concatenate([se[0], se[1], padidx(DSP - 2 * ES)]
                           ).reshape(DSP // CH, CH)
    cdF = jnp.concatenate([config_dst.astype(i32), padidx(24)]).reshape(8, CH)
    cdS = jnp.concatenate([sampled_config_dst.astype(i32), padidx(24)]
                          ).reshape(8, CH)
    cdpad = jnp.pad(config_dst.astype(i32), (0, 24)).reshape(32, 32)
    P = jnp.pad(jnp.reshape(nconfig_feats, (NCF, C * CFD)),
                ((0, 24), (0, CFW - C * CFD)))
    z128 = jnp.zeros((NP, HW), _f32)
    z80 = jnp.zeros((NP, RW // 2), _f32)
    z96 = jnp.zeros((NP, CFW), _f32)
    z16 = jnp.zeros((NP, 16), _f32)
    idsf = op_ids.astype(i32)[:, None]
    self_f = selected.astype(_f32)[:, None]
    gof = graph_id_op.astype(i32)[:, None]
    gic = jnp.pad(graph_id_config.astype(i32), (0, 24),
                  constant_values=G)[:, None]
    Wcp, Wf, We = pre_W1[:CFD], pre_W1[CFD:CFD + OPF], pre_W1[CFD + OPF:]
    b_pre1 = pre_b1[None, :]
    b_pre2 = pre_b2[None, :]
    b_g11 = gc1_b1[None, :]
    b_g12 = gc1_b2[None, :]
    b_g21 = gc2_b1[None, :]
    b_g22 = gc2_b2[None, :]

    sds = jax.ShapeDtypeStruct

    sc_pro = pl.kernel(
        _sc_pro_body,
        out_type=[sds((NP, CFW), _f32), sds((NP, CFW), _f32),
                  sds((2, NP, 16), _f32), sds((2, NP, 16), _f32)],
        mesh=mesh,
        scratch_types=[pltpu.VMEM_SHARED((NP, CFW), _f32),
                       pltpu.VMEM_SHARED((NP, 16), _f32),
                       pltpu.VMEM((CH, CFW), _f32),
                       pltpu.VMEM((CH, 16), _f32),
                       pltpu.VMEM((DFP // CH // NW, CH), i32),
                       pltpu.VMEM((8, CH), i32)],
        compiler_params=scparams)
    cfF, cfS, degF, degS = sc_pro(idxF, idxS, cdF, cdS, P, z96, z16)

    def fullspec(shp):
        return pl.BlockSpec(shp, lambda i: tuple(0 for _ in shp))

    def blkspec(r):
        return pl.BlockSpec((BLK, r), lambda i: (i, 0))

    degspec = pl.BlockSpec((2, BLK, 16), lambda i: (0, i, 0))
    HX = RW // 2

    def tc_pre(cf, deg):
        return pl.pallas_call(
            _tc_pre_body,
            grid=(NBLK,),
            in_specs=[blkspec(OPF), blkspec(1), blkspec(CFW), degspec,
                      fullspec((VOCAB, HID)), fullspec((OPF, HID)),
                      fullspec((HID, HID)), fullspec((CFD, HID)),
                      fullspec((1, HID)), fullspec((HID, HID)),
                      fullspec((1, HID))],
            out_specs=[blkspec(RW), blkspec(HW), blkspec(HW), blkspec(1)],
            out_shape=[sds((NP, RW), _f32), sds((NP, HW), _f32),
                       sds((NP, HW), _f32), sds((NP, 1), _f32)],
        )(op_feats, idsf, cf, deg,
          emb, Wf, We, Wcp, b_pre1, pre_W2, b_pre2)

    def adj_call(w, yA, yB, si, di, zz):
        return pl.kernel(
            _sc_adj_body,
            out_type=[sds((NP, w), _f32), sds((NP, w), _f32)],
            mesh=mesh,
            scratch_types=[pltpu.VMEM_SHARED((NP, w), _f32),
                           pltpu.VMEM((IBLK, GCH), i32),
                           pltpu.VMEM((IBLK, GCH), i32),
                           pltpu.VMEM((GCH, w), _f32),
                           pltpu.VMEM((GCH, w), _f32),
                           pltpu.SemaphoreType.DMA,
                           pltpu.SemaphoreType.DMA],
            compiler_params=scparams)(yA, yB, si, di, zz)

    def tc_mid1(x0, yA, yB, pA, pB, dis):
        return pl.pallas_call(
            _tc_mid1_body,
            grid=(NBLK,),
            in_specs=[blkspec(RW),
                      blkspec(HW), blkspec(HW), blkspec(HW), blkspec(HW),
                      blkspec(1),
                      fullspec((YD, HID)), fullspec((1, HID)),
                      fullspec((HID, HID)), fullspec((1, HID))],
            out_specs=[blkspec(RW), blkspec(HX), blkspec(HX)],
            out_shape=[sds((NP, RW), _f32), sds((NP, HX), _f32),
                       sds((NP, HX), _f32)],
        )(x0, yA, yB, pA, pB, dis, gc1_W1, b_g11, gc1_W2, b_g12)

    x0F, y1AF, y1BF, disF = tc_pre(cfF, degF)
    p1AF, p1BF = adj_call(HW, y1AF, y1BF, sF, dF, z128)
    x0S, y1AS, y1BS, disS = tc_pre(cfS, degS)
    p1AS, p1BS = adj_call(HW, y1AS, y1BS, sS, dS, z128)
    x1F, y2AF, y2BF = tc_mid1(x0F, y1AF, y1BF, p1AF, p1BF, disF)
    p2AF, p2BF = adj_call(HX, y2AF, y2BF, sF, dF, z80)
    x1S, y2AS, y2BS = tc_mid1(x0S, y1AS, y1BS, p1AS, p1BS, disS)
    p2AS, p2BS = adj_call(HX, y2AS, y2BS, sS, dS, z80)

    x, pooled, counts = pl.pallas_call(
        _tc_mid2_body,
        grid=(NBLK,),
        in_specs=[blkspec(RW), blkspec(RW),
                  blkspec(HW), blkspec(HW), blkspec(HW), blkspec(HW),
                  blkspec(HX), blkspec(HX), blkspec(HX), blkspec(HX),
                  blkspec(HX), blkspec(HX), blkspec(HX), blkspec(HX),
                  blkspec(1), blkspec(1), blkspec(1), blkspec(1),
                  fullspec((YD, HID)), fullspec((1, HID)),
                  fullspec((HID, HID)), fullspec((1, HID))],
        out_specs=[blkspec(RW),
                   pl.BlockSpec((G, RW), lambda i: (0, 0)),
                   pl.BlockSpec((G, 1), lambda i: (0, 0))],
        out_shape=[sds((NP, RW), _f32), sds((G, RW), _f32), sds((G, 1), _f32)],
    )(x1F, x1S, y1AF, y1AS, p1AF, p1AS,
      y2AF, y2BF, y2AS, y2BS, p2AF, p2BF, p2AS, p2BS,
      disF, disS, self_f, gof,
      gc2_W1, b_g21, gc2_W2, b_g22)

    sc_epi = pl.kernel(
        _sc_epi_body,
        out_type=sds((1024, RW), _f32),
        mesh=mesh,
        scratch_types=[pltpu.VMEM((32, 32), i32),
                       pltpu.VMEM((32, RW), _f32),
                       pltpu.SemaphoreType.DMA],
        compiler_params=scparams)
    cfgx = sc_epi(x, cdpad)

    out8 = pl.pallas_call(
        _tc_final_body,
        grid=(1,),
        in_specs=[fullspec((G, RW)), fullspec((G, 1)),
                  fullspec((1024, RW)), fullspec((1024, 1)),
                  fullspec((3 * HID, HID)), fullspec((HID, 1))],
        out_specs=[fullspec((G, 128))],
        out_shape=[sds((G, 128), _f32)],
    )(pooled, counts, cfgx, gic, post_W1, post_W2)[0]

    return out8[:, :C]


# speedup vs baseline: 178.0533x; 1.0988x over previous
import jax
import jax.numpy as jnp
from jax import lax
from jax.experimental import pallas as pl
from jax.experimental.pallas import tpu as pltpu
from jax.experimental.pallas import tpu_sc as plsc

N = 10000
C = 5
HID = 32
RW = C * HID
YD = 50
YW = 256
HW = 128
CFD = 18
CFW = 96
G = 8
VOCAB = 120
OPF = 140
EF = 320000
ES = 160000
NCF = 1000
CH = 128
NW = 32
EFP = 327680
ESP = 163840
DFP = 655360
DSP = 327680
NP = 10112
NTILES = 16
STRIPE = NP // NTILES
ALPHA = 0.2
BLK = 400
NBLK = N // BLK
GCH = 128
IBLK = 32

_f32 = jnp.float32


def _leaky(x):
    return jnp.where(x > 0, x, ALPHA * x)


def _dot(a, b):
    return jnp.dot(a, b, precision=lax.Precision.DEFAULT,
                   preferred_element_type=_f32)


def _dotx(a, b):
    return jnp.dot(a, b, precision=lax.Precision.HIGHEST,
                   preferred_element_type=_f32)


def _dotx_t(a, b):
    return lax.dot_general(a, b, (((0,), (0,)), ((), ())),
                           precision=lax.Precision.HIGHEST,
                           preferred_element_type=_f32)


def _l2n(y):
    return y * lax.rsqrt(jnp.maximum(jnp.sum(y * y, axis=-1, keepdims=True),
                                     1e-12))


def _sc_pro_body(idxF, idxS, cdF, cdS, P, z96, z16,
                 cfF, cfS, degF, degS,
                 acc_cf, acc_deg, pbuf, ones, idxb, cdv):
    c = lax.axis_index("c")
    s = lax.axis_index("s")
    wid = c * NTILES + s
    row0 = s * STRIPE

    def fill(j, carry):
        ones[j] = jnp.ones((16,), _f32)
        return carry
    lax.fori_loop(0, CH, fill, 0)

    pltpu.sync_copy(z96.at[pl.ds(row0, STRIPE)], acc_cf.at[pl.ds(row0, STRIPE)])
    plsc.subcore_barrier()

    @pl.when((c == 0) & (s == 0))
    def _():
        pltpu.sync_copy(cdF, cdv)

    @pl.when((c == 1) & (s == 0))
    def _():
        pltpu.sync_copy(cdS, cdv)

    @pl.when(s == 0)
    def _():
        def cbody(j, carry):
            pltpu.sync_copy(P.at[pl.ds(j * CH, CH)], pbuf)
            pltpu.sync_copy(pbuf, acc_cf.at[cdv.at[j]], add=True)
            return carry
        lax.fori_loop(0, NCF // CH + 1, cbody, 0)

    plsc.subcore_barrier()

    @pl.when(c == 0)
    def _():
        pltpu.sync_copy(acc_cf.at[pl.ds(row0, STRIPE)],
                        cfF.at[pl.ds(row0, STRIPE)])

    @pl.when(c == 1)
    def _():
        pltpu.sync_copy(acc_cf.at[pl.ds(row0, STRIPE)],
                        cfS.at[pl.ds(row0, STRIPE)])

    for idx_hbm, nch, out in ((idxF, DFP // CH // NW, degF),
                              (idxS, DSP // CH // NW, degS)):
        pltpu.sync_copy(z16.at[pl.ds(row0, STRIPE)],
                        acc_deg.at[pl.ds(row0, STRIPE)])
        plsc.subcore_barrier()
        pltpu.sync_copy(idx_hbm.at[pl.ds(wid * nch, nch)],
                        idxb.at[pl.ds(0, nch)])

        def body(j, carry):
            pltpu.sync_copy(ones, acc_deg.at[idxb.at[j]], add=True)
            return carry
        lax.fori_loop(0, nch, body, 0)
        plsc.subcore_barrier()
        pltpu.sync_copy(acc_deg.at[pl.ds(row0, STRIPE)],
                        out.at[c].at[pl.ds(row0, STRIPE)])
        plsc.subcore_barrier()


def _sc_adj_body(yA, yB, si, di, zz,
                 aA, aB,
                 acc, sidx, didx, rowsA, rowsB, semA, semB):
    c = lax.axis_index("c")
    s = lax.axis_index("s")
    row0 = s * STRIPE

    def run(ys, ept, out):
        pltpu.sync_copy(zz.at[pl.ds(row0, STRIPE)],
                        acc.at[pl.ds(row0, STRIPE)])
        plsc.subcore_barrier()
        base_row = s * (ept // GCH)

        def outer(g, carry):
            pltpu.sync_copy(si.at[pl.ds(base_row + g * IBLK, IBLK)], sidx)
            pltpu.sync_copy(di.at[pl.ds(base_row + g * IBLK, IBLK)], didx)
            pltpu.async_copy(ys.at[sidx.at[0]], rowsA, semA)

            def body(k, carry2):
                pltpu.async_copy(ys.at[didx.at[k]], rowsB, semB)
                pltpu.make_async_copy(ys.at[sidx.at[k]], rowsA, semA).wait()
                pltpu.sync_copy(rowsA, acc.at[didx.at[k]], add=True)
                kn = jnp.minimum(k + 1, IBLK - 1)
                pltpu.async_copy(ys.at[sidx.at[kn]], rowsA, semA)
                pltpu.make_async_copy(ys.at[didx.at[k]], rowsB, semB).wait()
                pltpu.sync_copy(rowsB, acc.at[sidx.at[k]], add=True)
                return carry2
            lax.fori_loop(0, IBLK, body, 0)
            pltpu.make_async_copy(ys.at[sidx.at[IBLK - 1]], rowsA, semA).wait()
            return carry
        lax.fori_loop(0, ept // GCH // IBLK, outer, 0)
        plsc.subcore_barrier()
        pltpu.sync_copy(acc.at[pl.ds(row0, STRIPE)],
                        out.at[pl.ds(row0, STRIPE)])
        plsc.subcore_barrier()

    ept = si.shape[0] * GCH // NTILES

    @pl.when(c == 0)
    def _():
        run(yA, ept, aA)

    @pl.when(c == 1)
    def _():
        run(yB, ept, aB)


def _sc_epi_body(x, cd, out, idxv, buf, sem):
    c = lax.axis_index("c")
    s = lax.axis_index("s")
    wid = c * NTILES + s
    pltpu.sync_copy(cd, idxv)
    pltpu.async_copy(x.at[idxv.at[wid]], buf, sem).wait()
    pltpu.sync_copy(buf, out.at[pl.ds(wid * 32, 32)])


def _ys_halves(dis, cfb, xcols):
    ys = dis * jnp.concatenate([cfb] + xcols, axis=1)
    return ys[:, :HW], ys[:, HW:]


def _a_cfg(acf, ax, k):
    return jnp.concatenate([acf[:, k * CFD:(k + 1) * CFD],
                            ax[:, k * HID:(k + 1) * HID]], axis=1)


def _tc_pre_body(opf, idsf, cf, dg,
                 emb, Wf, We, Wcp, b1, W2, b2,
                 x0o, yAo, yBo, diso):
    oh = (idsf[...] == lax.broadcasted_iota(jnp.int32, (BLK, VOCAB), 1))
    e = _dotx(oh.astype(_f32), emb[...])
    nfb = _dot(opf[...], Wf[...]) + _dot(e, We[...]) + b1[...]
    dgv = dg[...]
    dis = lax.rsqrt(1.0 + dgv[0, :, 0:1] + dgv[1, :, 0:1])
    diso[...] = dis
    cfb = cf[...] * 100.0
    xcols = []
    for k in range(C):
        cfc = cfb[:, k * CFD:(k + 1) * CFD]
        t = _leaky(_dot(cfc, Wcp[...]) + nfb)
        x0c = _leaky(_dot(t, W2[...]) + b2[...])
        xcols.append(x0c)
        x0o[:, k * HID:(k + 1) * HID] = x0c
    yA, yB = _ys_halves(dis, cfb, xcols)
    yAo[...] = yA
    yBo[...] = yB


def _tc_mid1_body(x0, yA, yB, pA, pB, dis,
                  W1, b1, W2, b2,
                  x1o, y2Ao, y2Bo):
    d = dis[...]
    full = jnp.concatenate([yA[...] + pA[...], yB[...] + pB[...]], axis=1)
    acf = d * full[:, :CFW]
    ax = d * full[:, CFW:]
    x0v = x0[...]
    xcols = []
    for k in range(C):
        ac = _a_cfg(acf, ax, k)
        h = _dot(_leaky(_dot(ac, W1[...]) + b1[...]), W2[...]) + b2[...]
        x1c = x0v[:, k * HID:(k + 1) * HID] + _leaky(h)
        xcols.append(x1c)
        x1o[:, k * HID:(k + 1) * HID] = x1c
    yx = d * jnp.concatenate(xcols, axis=1)
    y2Ao[...] = yx[:, :RW // 2]
    y2Bo[...] = yx[:, RW // 2:]


def _tc_mid2_body(x1F, x1S, y1AF, y1AS, p1AF, p1AS,
                  y2AF, y2BF, y2AS, y2BS, pAF, pBF, pAS, pBS,
                  disF, disS, self_f, gof,
                  W1, b1, W2, b2,
                  x, pooled, counts):
    xs = []
    for x1, y1A, p1A, y2A, y2B, pA, pB, dis in (
            (x1F, y1AF, p1AF, y2AF, y2BF, pAF, pBF, disF),
            (x1S, y1AS, p1AS, y2AS, y2BS, pAS, pBS, disS)):
        d = dis[...]
        acf = d * (y1A[...][:, :CFW] + p1A[...][:, :CFW])
        ax = d * jnp.concatenate([y2A[...] + pA[...], y2B[...] + pB[...]],
                                 axis=1)
        x1v = x1[...]
        cols = []
        for k in range(C):
            ac = _a_cfg(acf, ax, k)
            h = _dot(_leaky(_dot(ac, W1[...]) + b1[...]), W2[...]) + b2[...]
            cols.append(x1v[:, k * HID:(k + 1) * HID] + _leaky(h))
        xs.append(jnp.concatenate(cols, axis=1))
    sel = self_f[...]
    xb = sel * xs[1] + (1.0 - sel) * xs[0]
    x[...] = xb
    oh = (gof[...] == lax.broadcasted_iota(jnp.int32, (BLK, G), 1)).astype(_f32)
    ps = _dotx_t(oh, xb)
    cnt = _dotx_t(oh, jnp.ones((BLK, 1), _f32))

    @pl.when(pl.program_id(0) == 0)
    def _():
        pooled[...] = ps
        counts[...] = cnt

    @pl.when(pl.program_id(0) != 0)
    def _():
        pooled[...] += ps
        counts[...] += cnt


def _tc_final_body(pooled, counts, cfgx, gidc, pW1, pW2, out):
    oh = (gidc[...] == lax.broadcasted_iota(jnp.int32, (1024, G), 1)).astype(_f32)
    pc = _dotx_t(oh, cfgx[...])
    ps = pooled[...]
    pm = ps / jnp.maximum(counts[...], 1.0)
    cols = []
    for k in range(C):
        sl = slice(k * HID, (k + 1) * HID)
        h = jnp.concatenate([pm[:, sl], _l2n(ps[:, sl]), _l2n(pc[:, sl])],
                            axis=1)
        cols.append(_dot(_leaky(_dot(h, pW1[...])), pW2[...]))
    cols.append(jnp.zeros((G, 128 - C), _f32))
    out[...] = jnp.concatenate(cols, axis=1)


def kernel(op_feats, nconfig_feats, emb, pre_W1, pre_b1, pre_W2, pre_b2,
           gc1_W1, gc1_b1, gc1_W2, gc1_b2, gc2_W1, gc2_b1, gc2_W2, gc2_b2,
           post_W1, post_W2, op_ids, selected, feed_edges, sampled_feed_edges,
           config_dst, sampled_config_dst, graph_id_op, graph_id_config):
    i32 = jnp.int32
    mesh = plsc.VectorSubcoreMesh(core_axis_name="c", subcore_axis_name="s")
    scparams = pltpu.CompilerParams(use_tc_tiling_on_sc=False)

    def padidx(n):
        return N + (jnp.arange(n, dtype=i32) % 16)

    fe = feed_edges.astype(i32)
    se = sampled_feed_edges.astype(i32)
    sF = jnp.concatenate([fe[0], padidx(EFP - EF)]).reshape(EFP // GCH, GCH)
    dF = jnp.concatenate([fe[1], padidx(EFP - EF)]).reshape(EFP // GCH, GCH)
    sS = jnp.concatenate([se[0], padidx(ESP - ES)]).reshape(ESP // GCH, GCH)
    dS = jnp.concatenate([se[1], padidx(ESP - ES)]).reshape(ESP // GCH, GCH)
    idxF = jnp.concatenate([fe[0], fe[1], padidx(DFP - 2 * EF)]
                           ).reshape(DFP // CH, CH)
    idxS = jnp.concatenate([se[0], se[1], padidx(DSP - 2 * ES)]
                           ).reshape(DSP // CH, CH)
    cdF = jnp.concatenate([config_dst.astype(i32), padidx(24)]).reshape(8, CH)
    cdS = jnp.concatenate([sampled_config_dst.astype(i32), padidx(24)]
                          ).reshape(8, CH)
    cdpad = jnp.pad(config_dst.astype(i32), (0, 24)).reshape(32, 32)
    P = jnp.pad(jnp.reshape(nconfig_feats, (NCF, C * CFD)),
                ((0, 24), (0, CFW - C * CFD)))
    z128 = jnp.zeros((NP, HW), _f32)
    z80 = jnp.zeros((NP, RW // 2), _f32)
    z96 = jnp.zeros((NP, CFW), _f32)
    z16 = jnp.zeros((NP, 16), _f32)
    idsf = op_ids.astype(i32)[:, None]
    self_f = selected.astype(_f32)[:, None]
    gof = graph_id_op.astype(i32)[:, None]
    gic = jnp.pad(graph_id_config.astype(i32), (0, 24),
                  constant_values=G)[:, None]
    Wcp, Wf, We = pre_W1[:CFD], pre_W1[CFD:CFD + OPF], pre_W1[CFD + OPF:]
    b_pre1 = pre_b1[None, :]
    b_pre2 = pre_b2[None, :]
    b_g11 = gc1_b1[None, :]
    b_g12 = gc1_b2[None, :]
    b_g21 = gc2_b1[None, :]
    b_g22 = gc2_b2[None, :]

    sds = jax.ShapeDtypeStruct

    sc_pro = pl.kernel(
        _sc_pro_body,
        out_type=[sds((NP, CFW), _f32), sds((NP, CFW), _f32),
                  sds((2, NP, 16), _f32), sds((2, NP, 16), _f32)],
        mesh=mesh,
        scratch_types=[pltpu.VMEM_SHARED((NP, CFW), _f32),
                       pltpu.VMEM_SHARED((NP, 16), _f32),
                       pltpu.VMEM((CH, CFW), _f32),
                       pltpu.VMEM((CH, 16), _f32),
                       pltpu.VMEM((DFP // CH // NW, CH), i32),
                       pltpu.VMEM((8, CH), i32)],
        compiler_params=scparams)
    cfF, cfS, degF, degS = sc_pro(idxF, idxS, cdF, cdS, P, z96, z16)

    def fullspec(shp):
        return pl.BlockSpec(shp, lambda i: tuple(0 for _ in shp))

    def blkspec(r):
        return pl.BlockSpec((BLK, r), lambda i: (i, 0))

    degspec = pl.BlockSpec((2, BLK, 16), lambda i: (0, i, 0))
    HX = RW // 2

    def tc_pre(cf, deg):
        return pl.pallas_call(
            _tc_pre_body,
            grid=(NBLK,),
            in_specs=[blkspec(OPF), blkspec(1), blkspec(CFW), degspec,
                      fullspec((VOCAB, HID)), fullspec((OPF, HID)),
                      fullspec((HID, HID)), fullspec((CFD, HID)),
                      fullspec((1, HID)), fullspec((HID, HID)),
                      fullspec((1, HID))],
            out_specs=[blkspec(RW), blkspec(HW), blkspec(HW), blkspec(1)],
            out_shape=[sds((NP, RW), _f32), sds((NP, HW), _f32),
                       sds((NP, HW), _f32), sds((NP, 1), _f32)],
        )(op_feats, idsf, cf, deg,
          emb, Wf, We, Wcp, b_pre1, pre_W2, b_pre2)

    def adj_call(w, yA, yB, si, di, zz):
        return pl.kernel(
            _sc_adj_body,
            out_type=[sds((NP, w), _f32), sds((NP, w), _f32)],
            mesh=mesh,
            scratch_types=[pltpu.VMEM_SHARED((NP, w), _f32),
                           pltpu.VMEM((IBLK, GCH), i32),
                           pltpu.VMEM((IBLK, GCH), i32),
                           pltpu.VMEM((GCH, w), _f32),
                           pltpu.VMEM((GCH, w), _f32),
                           pltpu.SemaphoreType.DMA,
                           pltpu.SemaphoreType.DMA],
            compiler_params=scparams)(yA, yB, si, di, zz)

    def tc_mid1(x0, yA, yB, pA, pB, dis):
        return pl.pallas_call(
            _tc_mid1_body,
            grid=(NBLK,),
            in_specs=[blkspec(RW),
                      blkspec(HW), blkspec(HW), blkspec(HW), blkspec(HW),
                      blkspec(1),
                      fullspec((YD, HID)), fullspec((1, HID)),
                      fullspec((HID, HID)), fullspec((1, HID))],
            out_specs=[blkspec(RW), blkspec(HX), blkspec(HX)],
            out_shape=[sds((NP, RW), _f32), sds((NP, HX), _f32),
                       sds((NP, HX), _f32)],
        )(x0, yA, yB, pA, pB, dis, gc1_W1, b_g11, gc1_W2, b_g12)

    x0F, y1AF, y1BF, disF = tc_pre(cfF, degF)
    p1AF, p1BF = adj_call(HW, y1AF, y1BF, sF, dF, z128)
    x0S, y1AS, y1BS, disS = tc_pre(cfS, degS)
    p1AS, p1BS = adj_call(HW, y1AS, y1BS, sS, dS, z128)
    x1F, y2AF, y2BF = tc_mid1(x0F, y1AF, y1BF, p1AF, p1BF, disF)
    p2AF, p2BF = adj_call(HX, y2AF, y2BF, sF, dF, z80)
    x1S, y2AS, y2BS = tc_mid1(x0S, y1AS, y1BS, p1AS, p1BS, disS)
    p2AS, p2BS = adj_call(HX, y2AS, y2BS, sS, dS, z80)

    x, pooled, counts = pl.pallas_call(
        _tc_mid2_body,
        grid=(NBLK,),
        in_specs=[blkspec(RW), blkspec(RW),
                  blkspec(HW), blkspec(HW), blkspec(HW), blkspec(HW),
                  blkspec(HX), blkspec(HX), blkspec(HX), blkspec(HX),
                  blkspec(HX), blkspec(HX), blkspec(HX), blkspec(HX),
                  blkspec(1), blkspec(1), blkspec(1), blkspec(1),
                  fullspec((YD, HID)), fullspec((1, HID)),
                  fullspec((HID, HID)), fullspec((1, HID))],
        out_specs=[blkspec(RW),
                   pl.BlockSpec((G, RW), lambda i: (0, 0)),
                   pl.BlockSpec((G, 1), lambda i: (0, 0))],
        out_shape=[sds((NP, RW), _f32), sds((G, RW), _f32), sds((G, 1), _f32)],
    )(x1F, x1S, y1AF, y1AS, p1AF, p1AS,
      y2AF, y2BF, y2AS, y2BS, p2AF, p2BF, p2AS, p2BS,
      disF, disS, self_f, gof,
      gc2_W1, b_g21, gc2_W2, b_g22)

    sc_epi = pl.kernel(
        _sc_epi_body,
        out_type=sds((1024, RW), _f32),
        mesh=mesh,
        scratch_types=[pltpu.VMEM((32, 32), i32),
                       pltpu.VMEM((32, RW), _f32),
                       pltpu.SemaphoreType.DMA],
        compiler_params=scparams)
    cfgx = sc_epi(x, cdpad)

    out8 = pl.pallas_call(
        _tc_final_body,
        grid=(1,),
        in_specs=[fullspec((G, RW)), fullspec((G, 1)),
                  fullspec((1024, RW)), fullspec((1024, 1)),
                  fullspec((3 * HID, HID)), fullspec((HID, 1))],
        out_specs=[fullspec((G, 128))],
        out_shape=[sds((G, 128), _f32)],
    )(pooled, counts, cfgx, gic, post_W1, post_W2)[0]

    return out8[:, :C]
